# Initial kernel scaffold; baseline (speedup 1.0000x reference)
import jax
import jax.numpy as jnp
from jax.experimental import pallas as pl


def kernel(state, edge_index, action, W1, b1, W2, b2, W3, b3, lin1W, lin1b, lin2W, lin2b, lin3W, lin3b):
    raise NotImplementedError("write your pallas kernel here")


# speedup vs baseline: 8.8735x; 8.8735x over previous
import functools

import jax
import jax.numpy as jnp
from jax import lax
from jax.experimental import pallas as pl
from jax.experimental.pallas import tpu as pltpu
from jax.experimental.pallas import tpu_sc as plsc

_N = 10000
_C = 128
_E = 320000
_NW = 32
_PER_W = _E // _NW
_CH = 80
_NCH = _PER_W // _CH

_mesh = plsc.VectorSubcoreMesh(core_axis_name="c", subcore_axis_name="s")


@functools.partial(
    pl.kernel,
    out_type=jax.ShapeDtypeStruct((2, _N), jnp.float32),
    scratch_types=[
        pltpu.VMEM_SHARED((_N,), jnp.float32),
        pltpu.VMEM((_CH,), jnp.float32),
        pltpu.VMEM((_CH,), jnp.int32),
    ],
    mesh=_mesh,
)
def _sc_degree(dst_hbm, zeros_hbm, ones_hbm, out_hbm, shared_deg, ones_v,
               didx_v):
    cid = lax.axis_index("c")
    sid = lax.axis_index("s")

    @pl.when(sid == 0)
    def _():
        pltpu.sync_copy(zeros_hbm, shared_deg)

    pltpu.sync_copy(ones_hbm, ones_v)
    plsc.subcore_barrier()
    base = (cid * 16 + sid) * _PER_W

    def body(i, _):
        pltpu.sync_copy(dst_hbm.at[pl.ds(base + i * _CH, _CH)], didx_v)
        pltpu.sync_copy(ones_v, shared_deg.at[didx_v], add=True)
        return 0

    lax.fori_loop(0, _NCH, body, 0)
    plsc.subcore_barrier()

    @pl.when(sid == 0)
    def _():
        pltpu.sync_copy(shared_deg, out_hbm.at[cid])


@functools.partial(
    pl.kernel,
    out_type=jax.ShapeDtypeStruct((2, _N, _C), jnp.float32),
    scratch_types=[
        pltpu.VMEM_SHARED((_N, _C), jnp.float32),
        pltpu.VMEM((_CH,), jnp.int32),
        pltpu.VMEM((_CH,), jnp.int32),
        pltpu.VMEM((_CH, _C), jnp.float32),
        pltpu.SemaphoreType.DMA,
    ],
    mesh=_mesh,
)
def _sc_segsum(ys_hbm, src_hbm, dst_hbm, zeros_hbm, out_hbm, shared_acc,
               sidx_v, didx_v, rows_v, sem):
    cid = lax.axis_index("c")
    sid = lax.axis_index("s")

    @pl.when(sid == 0)
    def _():
        pltpu.sync_copy(zeros_hbm, shared_acc)

    plsc.subcore_barrier()
    base = (cid * 16 + sid) * _PER_W

    def body(i, _):
        off = base + i * _CH
        pltpu.sync_copy(src_hbm.at[pl.ds(off, _CH)], sidx_v)
        pltpu.async_copy(ys_hbm.at[sidx_v], rows_v, sem).wait()
        pltpu.sync_copy(dst_hbm.at[pl.ds(off, _CH)], didx_v)
        pltpu.sync_copy(rows_v, shared_acc.at[didx_v], add=True)
        return 0

    lax.fori_loop(0, _NCH, body, 0)
    plsc.subcore_barrier()

    @pl.when(sid == 0)
    def _():
        pltpu.sync_copy(shared_acc, out_hbm.at[cid])


def _dis_body(deg_ref, out_ref):
    d = deg_ref[0:1, :] + deg_ref[1:2, :] + 1.0
    out_ref[...] = lax.rsqrt(d)


def _tc_dis(deg2):
    return pl.pallas_call(
        _dis_body,
        out_shape=jax.ShapeDtypeStruct((1, _N), jnp.float32),
    )(deg2)


_R = 2000


def _project_body(h_ref, w_ref, dis_ref, xw_ref, ys_ref):
    xw = jnp.dot(h_ref[...], w_ref[...], preferred_element_type=jnp.float32)
    xw_ref[...] = xw
    ys_ref[...] = xw * dis_ref[...]


def _tc_project(h, w, dis_col):
    grid = (_N // _R,)
    return pl.pallas_call(
        _project_body,
        grid=grid,
        in_specs=[
            pl.BlockSpec((_R, _C), lambda i: (i, 0)),
            pl.BlockSpec((_C, _C), lambda i: (0, 0)),
            pl.BlockSpec((_R, 1), lambda i: (i, 0)),
        ],
        out_specs=[
            pl.BlockSpec((_R, _C), lambda i: (i, 0)),
            pl.BlockSpec((_R, _C), lambda i: (i, 0)),
        ],
        out_shape=[
            jax.ShapeDtypeStruct((_N, _C), jnp.float32),
            jax.ShapeDtypeStruct((_N, _C), jnp.float32),
        ],
    )(h, w, dis_col)


def _layer_mid_body(a0_ref, a1_ref, xw_ref, dis_ref, b_ref, w_ref,
                    out_ref, xwn_ref, ysn_ref):
    d = dis_ref[...]
    agg = (a0_ref[...] + a1_ref[...]) * d + xw_ref[...] * (d * d) + b_ref[...]
    out = jnp.maximum(agg, 0.0)
    out_ref[...] = out
    xwn = jnp.dot(out, w_ref[...], preferred_element_type=jnp.float32)
    xwn_ref[...] = xwn
    ysn_ref[...] = xwn * d


def _tc_layer_mid(a0, a1, xw, dis_col, b_row, w_next):
    grid = (_N // _R,)
    return pl.pallas_call(
        _layer_mid_body,
        grid=grid,
        in_specs=[
            pl.BlockSpec((_R, _C), lambda i: (i, 0)),
            pl.BlockSpec((_R, _C), lambda i: (i, 0)),
            pl.BlockSpec((_R, _C), lambda i: (i, 0)),
            pl.BlockSpec((_R, 1), lambda i: (i, 0)),
            pl.BlockSpec((1, _C), lambda i: (0, 0)),
            pl.BlockSpec((_C, _C), lambda i: (0, 0)),
        ],
        out_specs=[
            pl.BlockSpec((_R, _C), lambda i: (i, 0)),
            pl.BlockSpec((_R, _C), lambda i: (i, 0)),
            pl.BlockSpec((_R, _C), lambda i: (i, 0)),
        ],
        out_shape=[
            jax.ShapeDtypeStruct((_N, _C), jnp.float32),
            jax.ShapeDtypeStruct((_N, _C), jnp.float32),
            jax.ShapeDtypeStruct((_N, _C), jnp.float32),
        ],
    )(a0, a1, xw, dis_col, b_row, w_next)


def _layer_last_body(a0_ref, a1_ref, xw_ref, dis_ref, b_ref, out_ref):
    d = dis_ref[...]
    agg = (a0_ref[...] + a1_ref[...]) * d + xw_ref[...] * (d * d) + b_ref[...]
    out_ref[...] = jnp.maximum(agg, 0.0)


def _tc_layer_last(a0, a1, xw, dis_col, b_row):
    grid = (_N // _R,)
    return pl.pallas_call(
        _layer_last_body,
        grid=grid,
        in_specs=[
            pl.BlockSpec((_R, _C), lambda i: (i, 0)),
            pl.BlockSpec((_R, _C), lambda i: (i, 0)),
            pl.BlockSpec((_R, _C), lambda i: (i, 0)),
            pl.BlockSpec((_R, 1), lambda i: (i, 0)),
            pl.BlockSpec((1, _C), lambda i: (0, 0)),
        ],
        out_specs=pl.BlockSpec((_R, _C), lambda i: (i, 0)),
        out_shape=jax.ShapeDtypeStruct((_N, _C), jnp.float32),
    )(a0, a1, xw, dis_col, b_row)


_G = _R // 8


def _readout_body(o1_ref, o2_ref, o3_ref, o4_ref, o5_ref, st_ref, act_ref,
                  l1_ref, la_ref, l1b_ref, l2_ref, l2b_ref, l3_ref, l3b_ref,
                  out_ref):
    f32 = jnp.float32
    x1 = jnp.dot(o1_ref[...], l1_ref[0], preferred_element_type=f32)
    x1 += jnp.dot(o2_ref[...], l1_ref[1], preferred_element_type=f32)
    x1 += jnp.dot(o3_ref[...], l1_ref[2], preferred_element_type=f32)
    x1 += jnp.dot(o4_ref[...], l1_ref[3], preferred_element_type=f32)
    x1 += jnp.dot(o5_ref[...], l1_ref[4], preferred_element_type=f32)
    x1 += jnp.dot(st_ref[...], l1_ref[5], preferred_element_type=f32)
    x1 += act_ref[...] * la_ref[...] + l1b_ref[...]
    x1 = jnp.maximum(x1, 0.0)
    x2 = jnp.maximum(
        jnp.dot(x1, l2_ref[...], preferred_element_type=f32) + l2b_ref[...],
        0.0)
    s = jnp.dot(x2, l3_ref[...], preferred_element_type=f32)
    rows = lax.broadcasted_iota(jnp.int32, (_G, _R), 1)
    grp = lax.broadcasted_iota(jnp.int32, (_G, _R), 0)
    pmat = jnp.where(rows // 8 == grp, 1.0, 0.0).astype(f32)
    res = jnp.dot(pmat, s, preferred_element_type=f32) + l3b_ref[...]
    out_ref[...] = res[None]


def _tc_readout(o1, o2, o3, o4, o5, st, act_col, l1s, la_row, l1b_row, l2t,
                l2b_row, l3col, l3b11):
    grid = (_N // _R,)
    rc = pl.BlockSpec((_R, _C), lambda i: (i, 0))
    return pl.pallas_call(
        _readout_body,
        grid=grid,
        in_specs=[
            rc, rc, rc, rc, rc, rc,
            pl.BlockSpec((_R, 1), lambda i: (i, 0)),
            pl.BlockSpec((6, _C, 32), lambda i: (0, 0, 0)),
            pl.BlockSpec((1, 32), lambda i: (0, 0)),
            pl.BlockSpec((1, 32), lambda i: (0, 0)),
            pl.BlockSpec((32, 32), lambda i: (0, 0)),
            pl.BlockSpec((1, 32), lambda i: (0, 0)),
            pl.BlockSpec((32, 1), lambda i: (0, 0)),
            pl.BlockSpec((1, 1), lambda i: (0, 0)),
        ],
        out_specs=pl.BlockSpec((1, _G, 1), lambda i: (i, 0, 0)),
        out_shape=jax.ShapeDtypeStruct((_N // _R, _G, 1), jnp.float32),
    )(o1, o2, o3, o4, o5, st, act_col, l1s, la_row, l1b_row, l2t, l2b_row,
      l3col, l3b11)


def kernel(state, edge_index, action, W1, b1, W2, b2, W3, b3, lin1W, lin1b,
           lin2W, lin2b, lin3W, lin3b):
    src = edge_index[0]
    dst = edge_index[1]
    zeros_n = jnp.zeros((_N,), jnp.float32)
    zeros_nc = jnp.zeros((_N, _C), jnp.float32)
    ones_ch = jnp.ones((_CH,), jnp.float32)

    deg2 = _sc_degree(dst, zeros_n, ones_ch)
    dis_row = _tc_dis(deg2)
    dis_col = dis_row.reshape(_N, 1)

    def segsum(ys):
        acc = _sc_segsum(ys, src, dst, zeros_nc)
        return acc[0], acc[1]

    xw1, ys1 = _tc_project(state, W1, dis_col)
    a0, a1 = segsum(ys1)
    out1, xw2, ys2 = _tc_layer_mid(a0, a1, xw1, dis_col, b1.reshape(1, _C), W2)
    a0, a1 = segsum(ys2)
    out2, xw3, ys3 = _tc_layer_mid(a0, a1, xw2, dis_col, b2.reshape(1, _C), W3)
    a0, a1 = segsum(ys3)
    out3, xw4, ys4 = _tc_layer_mid(a0, a1, xw3, dis_col, b3.reshape(1, _C), W3)
    a0, a1 = segsum(ys4)
    out4, xw5, ys5 = _tc_layer_mid(a0, a1, xw4, dis_col, b3.reshape(1, _C), W3)
    a0, a1 = segsum(ys5)
    out5 = _tc_layer_last(a0, a1, xw5, dis_col, b3.reshape(1, _C))

    l1s = jnp.stack([
        lin1W[:, 0 * _C:1 * _C].T, lin1W[:, 1 * _C:2 * _C].T,
        lin1W[:, 2 * _C:3 * _C].T, lin1W[:, 3 * _C:4 * _C].T,
        lin1W[:, 4 * _C:5 * _C].T, lin1W[:, 5 * _C:6 * _C].T,
    ])
    la_row = lin1W[:, 6 * _C].reshape(1, 32)
    act_col = action.reshape(_N, 1)
    y = _tc_readout(out1, out2, out3, out4, out5, state, act_col, l1s,
                    la_row, lin1b.reshape(1, 32), lin2W.T,
                    lin2b.reshape(1, 32), lin3W.T, lin3b.reshape(1, 1))
    return y.reshape(_N // 8)


# speedup vs baseline: 13.7236x; 1.5466x over previous
import functools

import jax
import jax.numpy as jnp
from jax import lax
from jax.experimental import pallas as pl
from jax.experimental.pallas import tpu as pltpu
from jax.experimental.pallas import tpu_sc as plsc

_N = 10000
_C = 128
_E = 320000
_NW = 32
_PER_W = _E // _NW
_CH = 80
_NCH = _PER_W // _CH
_RPT = 624

_mesh = plsc.VectorSubcoreMesh(core_axis_name="c", subcore_axis_name="s")


@functools.partial(
    pl.kernel,
    out_type=jax.ShapeDtypeStruct((2, _N), jnp.float32),
    scratch_types=[
        pltpu.VMEM_SHARED((_N,), jnp.float32),
        pltpu.VMEM((_CH,), jnp.float32),
        pltpu.VMEM((_CH,), jnp.int32),
        pltpu.VMEM((_CH,), jnp.int32),
        pltpu.SemaphoreType.DMA,
        pltpu.SemaphoreType.DMA,
    ],
    mesh=_mesh,
)
def _sc_degree(dst_hbm, zeros_hbm, ones_hbm, out_hbm, shared_deg, ones_v,
               didx_a, didx_b, sem_da, sem_db):
    cid = lax.axis_index("c")
    sid = lax.axis_index("s")
    base = (cid * 16 + sid) * _PER_W

    @pl.when(sid == 0)
    def _():
        pltpu.sync_copy(zeros_hbm, shared_deg)

    pltpu.sync_copy(ones_hbm, ones_v)
    plsc.subcore_barrier()

    def body(j, _):
        o0 = base + 2 * j * _CH
        h_a = pltpu.async_copy(dst_hbm.at[pl.ds(o0, _CH)], didx_a, sem_da)
        h_b = pltpu.async_copy(dst_hbm.at[pl.ds(o0 + _CH, _CH)], didx_b,
                               sem_db)
        h_a.wait()
        pltpu.sync_copy(ones_v, shared_deg.at[didx_a], add=True)
        h_b.wait()
        pltpu.sync_copy(ones_v, shared_deg.at[didx_b], add=True)
        return 0

    lax.fori_loop(0, _NCH // 2, body, 0)
    h_t = pltpu.async_copy(dst_hbm.at[pl.ds(base + (_NCH - 1) * _CH, _CH)],
                           didx_a, sem_da)
    h_t.wait()
    pltpu.sync_copy(ones_v, shared_deg.at[didx_a], add=True)
    plsc.subcore_barrier()

    @pl.when(sid == 0)
    def _():
        pltpu.sync_copy(shared_deg, out_hbm.at[cid])


@functools.partial(
    pl.kernel,
    out_type=jax.ShapeDtypeStruct((2, _N, _C), jnp.float32),
    scratch_types=[
        pltpu.VMEM_SHARED((_N, _C), jnp.float32),
        pltpu.VMEM((_CH,), jnp.int32),
        pltpu.VMEM((_CH,), jnp.int32),
        pltpu.VMEM((_CH,), jnp.int32),
        pltpu.VMEM((_CH,), jnp.int32),
        pltpu.VMEM((_CH, _C), jnp.float32),
        pltpu.VMEM((_CH, _C), jnp.float32),
        pltpu.SemaphoreType.DMA,
        pltpu.SemaphoreType.DMA,
        pltpu.SemaphoreType.DMA,
        pltpu.SemaphoreType.DMA,
        pltpu.SemaphoreType.DMA,
        pltpu.SemaphoreType.DMA,
    ],
    mesh=_mesh,
)
def _sc_segsum(ys_hbm, src_hbm, dst_hbm, zeros_hbm, out_hbm, shared_acc,
               sidx_a, sidx_b, didx_a, didx_b, rows_a, rows_b, sem_sa,
               sem_sb, sem_da, sem_db, sem_a, sem_b):
    cid = lax.axis_index("c")
    sid = lax.axis_index("s")
    w = cid * 16 + sid

    @pl.when(sid < 15)
    def _():
        pltpu.sync_copy(zeros_hbm.at[pl.ds(sid * _RPT, _RPT)],
                        shared_acc.at[pl.ds(sid * _RPT, _RPT)])

    @pl.when(sid == 15)
    def _():
        pltpu.sync_copy(zeros_hbm.at[pl.ds(15 * _RPT, _N - 15 * _RPT)],
                        shared_acc.at[pl.ds(15 * _RPT, _N - 15 * _RPT)])

    base = (cid * 16 + sid) * _PER_W
    plsc.subcore_barrier()

    def body(j, _):
        o0 = base + 2 * j * _CH
        o1 = o0 + _CH
        hs_a = pltpu.async_copy(src_hbm.at[pl.ds(o0, _CH)], sidx_a, sem_sa)
        hs_b = pltpu.async_copy(src_hbm.at[pl.ds(o1, _CH)], sidx_b, sem_sb)
        hd_a = pltpu.async_copy(dst_hbm.at[pl.ds(o0, _CH)], didx_a, sem_da)
        hd_b = pltpu.async_copy(dst_hbm.at[pl.ds(o1, _CH)], didx_b, sem_db)
        hs_a.wait()
        h_a = pltpu.async_copy(ys_hbm.at[sidx_a], rows_a, sem_a)
        hs_b.wait()
        h_b = pltpu.async_copy(ys_hbm.at[sidx_b], rows_b, sem_b)
        h_a.wait()
        hd_a.wait()
        pltpu.sync_copy(rows_a, shared_acc.at[didx_a], add=True)
        h_b.wait()
        hd_b.wait()
        pltpu.sync_copy(rows_b, shared_acc.at[didx_b], add=True)
        return 0

    lax.fori_loop(0, _NCH // 2, body, 0)
    o_t = base + (_NCH - 1) * _CH
    hs_t = pltpu.async_copy(src_hbm.at[pl.ds(o_t, _CH)], sidx_a, sem_sa)
    hd_t = pltpu.async_copy(dst_hbm.at[pl.ds(o_t, _CH)], didx_a, sem_da)
    hs_t.wait()
    h_t = pltpu.async_copy(ys_hbm.at[sidx_a], rows_a, sem_a)
    h_t.wait()
    hd_t.wait()
    pltpu.sync_copy(rows_a, shared_acc.at[didx_a], add=True)
    plsc.subcore_barrier()

    @pl.when(sid < 15)
    def _():
        pltpu.sync_copy(shared_acc.at[pl.ds(sid * _RPT, _RPT)],
                        out_hbm.at[cid, pl.ds(sid * _RPT, _RPT)])

    @pl.when(sid == 15)
    def _():
        pltpu.sync_copy(shared_acc.at[pl.ds(15 * _RPT, _N - 15 * _RPT)],
                        out_hbm.at[cid, pl.ds(15 * _RPT, _N - 15 * _RPT)])


def _dis_body(deg_ref, out_ref):
    d = deg_ref[0:1, :] + deg_ref[1:2, :] + 1.0
    out_ref[...] = lax.rsqrt(d)


def _tc_dis(deg2):
    return pl.pallas_call(
        _dis_body,
        out_shape=jax.ShapeDtypeStruct((1, _N), jnp.float32),
    )(deg2)


_R = 2000


def _project_body(h_ref, w_ref, dis_ref, xw_ref, ys_ref):
    xw = jnp.dot(h_ref[...], w_ref[...], preferred_element_type=jnp.float32)
    xw_ref[...] = xw
    ys_ref[...] = xw * dis_ref[...]


def _tc_project(h, w, dis_col):
    grid = (_N // _R,)
    return pl.pallas_call(
        _project_body,
        grid=grid,
        in_specs=[
            pl.BlockSpec((_R, _C), lambda i: (i, 0)),
            pl.BlockSpec((_C, _C), lambda i: (0, 0)),
            pl.BlockSpec((_R, 1), lambda i: (i, 0)),
        ],
        out_specs=[
            pl.BlockSpec((_R, _C), lambda i: (i, 0)),
            pl.BlockSpec((_R, _C), lambda i: (i, 0)),
        ],
        out_shape=[
            jax.ShapeDtypeStruct((_N, _C), jnp.float32),
            jax.ShapeDtypeStruct((_N, _C), jnp.float32),
        ],
    )(h, w, dis_col)


def _layer_mid_body(a0_ref, a1_ref, xw_ref, dis_ref, b_ref, w_ref,
                    out_ref, xwn_ref, ysn_ref):
    d = dis_ref[...]
    agg = (a0_ref[...] + a1_ref[...]) * d + xw_ref[...] * (d * d) + b_ref[...]
    out = jnp.maximum(agg, 0.0)
    out_ref[...] = out
    xwn = jnp.dot(out, w_ref[...], preferred_element_type=jnp.float32)
    xwn_ref[...] = xwn
    ysn_ref[...] = xwn * d


def _tc_layer_mid(a0, a1, xw, dis_col, b_row, w_next):
    grid = (_N // _R,)
    return pl.pallas_call(
        _layer_mid_body,
        grid=grid,
        in_specs=[
            pl.BlockSpec((_R, _C), lambda i: (i, 0)),
            pl.BlockSpec((_R, _C), lambda i: (i, 0)),
            pl.BlockSpec((_R, _C), lambda i: (i, 0)),
            pl.BlockSpec((_R, 1), lambda i: (i, 0)),
            pl.BlockSpec((1, _C), lambda i: (0, 0)),
            pl.BlockSpec((_C, _C), lambda i: (0, 0)),
        ],
        out_specs=[
            pl.BlockSpec((_R, _C), lambda i: (i, 0)),
            pl.BlockSpec((_R, _C), lambda i: (i, 0)),
            pl.BlockSpec((_R, _C), lambda i: (i, 0)),
        ],
        out_shape=[
            jax.ShapeDtypeStruct((_N, _C), jnp.float32),
            jax.ShapeDtypeStruct((_N, _C), jnp.float32),
            jax.ShapeDtypeStruct((_N, _C), jnp.float32),
        ],
    )(a0, a1, xw, dis_col, b_row, w_next)


def _layer_last_body(a0_ref, a1_ref, xw_ref, dis_ref, b_ref, out_ref):
    d = dis_ref[...]
    agg = (a0_ref[...] + a1_ref[...]) * d + xw_ref[...] * (d * d) + b_ref[...]
    out_ref[...] = jnp.maximum(agg, 0.0)


def _tc_layer_last(a0, a1, xw, dis_col, b_row):
    grid = (_N // _R,)
    return pl.pallas_call(
        _layer_last_body,
        grid=grid,
        in_specs=[
            pl.BlockSpec((_R, _C), lambda i: (i, 0)),
            pl.BlockSpec((_R, _C), lambda i: (i, 0)),
            pl.BlockSpec((_R, _C), lambda i: (i, 0)),
            pl.BlockSpec((_R, 1), lambda i: (i, 0)),
            pl.BlockSpec((1, _C), lambda i: (0, 0)),
        ],
        out_specs=pl.BlockSpec((_R, _C), lambda i: (i, 0)),
        out_shape=jax.ShapeDtypeStruct((_N, _C), jnp.float32),
    )(a0, a1, xw, dis_col, b_row)


_G = _R // 8


def _readout_body(o1_ref, o2_ref, o3_ref, o4_ref, o5_ref, st_ref, act_ref,
                  l1_ref, la_ref, l1b_ref, l2_ref, l2b_ref, l3_ref, l3b_ref,
                  out_ref):
    f32 = jnp.float32
    x1 = jnp.dot(o1_ref[...], l1_ref[0], preferred_element_type=f32)
    x1 += jnp.dot(o2_ref[...], l1_ref[1], preferred_element_type=f32)
    x1 += jnp.dot(o3_ref[...], l1_ref[2], preferred_element_type=f32)
    x1 += jnp.dot(o4_ref[...], l1_ref[3], preferred_element_type=f32)
    x1 += jnp.dot(o5_ref[...], l1_ref[4], preferred_element_type=f32)
    x1 += jnp.dot(st_ref[...], l1_ref[5], preferred_element_type=f32)
    x1 += act_ref[...] * la_ref[...] + l1b_ref[...]
    x1 = jnp.maximum(x1, 0.0)
    x2 = jnp.maximum(
        jnp.dot(x1, l2_ref[...], preferred_element_type=f32) + l2b_ref[...],
        0.0)
    s = jnp.dot(x2, l3_ref[...], preferred_element_type=f32)
    rows = lax.broadcasted_iota(jnp.int32, (_G, _R), 1)
    grp = lax.broadcasted_iota(jnp.int32, (_G, _R), 0)
    pmat = jnp.where(rows // 8 == grp, 1.0, 0.0).astype(f32)
    res = jnp.dot(pmat, s, preferred_element_type=f32) + l3b_ref[...]
    out_ref[...] = res[None]


def _tc_readout(o1, o2, o3, o4, o5, st, act_col, l1s, la_row, l1b_row, l2t,
                l2b_row, l3col, l3b11):
    grid = (_N // _R,)
    rc = pl.BlockSpec((_R, _C), lambda i: (i, 0))
    return pl.pallas_call(
        _readout_body,
        grid=grid,
        in_specs=[
            rc, rc, rc, rc, rc, rc,
            pl.BlockSpec((_R, 1), lambda i: (i, 0)),
            pl.BlockSpec((6, _C, 32), lambda i: (0, 0, 0)),
            pl.BlockSpec((1, 32), lambda i: (0, 0)),
            pl.BlockSpec((1, 32), lambda i: (0, 0)),
            pl.BlockSpec((32, 32), lambda i: (0, 0)),
            pl.BlockSpec((1, 32), lambda i: (0, 0)),
            pl.BlockSpec((32, 1), lambda i: (0, 0)),
            pl.BlockSpec((1, 1), lambda i: (0, 0)),
        ],
        out_specs=pl.BlockSpec((1, _G, 1), lambda i: (i, 0, 0)),
        out_shape=jax.ShapeDtypeStruct((_N // _R, _G, 1), jnp.float32),
    )(o1, o2, o3, o4, o5, st, act_col, l1s, la_row, l1b_row, l2t, l2b_row,
      l3col, l3b11)


def kernel(state, edge_index, action, W1, b1, W2, b2, W3, b3, lin1W, lin1b,
           lin2W, lin2b, lin3W, lin3b):
    src = edge_index[0]
    dst = edge_index[1]
    zeros_n = jnp.zeros((_N,), jnp.float32)
    zeros_nc = jnp.zeros((_N, _C), jnp.float32)
    ones_ch = jnp.ones((_CH,), jnp.float32)

    deg2 = _sc_degree(dst, zeros_n, ones_ch)
    dis_row = _tc_dis(deg2)
    dis_col = dis_row.reshape(_N, 1)

    def segsum(ys):
        acc = _sc_segsum(ys, src, dst, zeros_nc)
        return acc[0], acc[1]

    xw1, ys1 = _tc_project(state, W1, dis_col)
    a0, a1 = segsum(ys1)
    out1, xw2, ys2 = _tc_layer_mid(a0, a1, xw1, dis_col, b1.reshape(1, _C), W2)
    a0, a1 = segsum(ys2)
    out2, xw3, ys3 = _tc_layer_mid(a0, a1, xw2, dis_col, b2.reshape(1, _C), W3)
    a0, a1 = segsum(ys3)
    out3, xw4, ys4 = _tc_layer_mid(a0, a1, xw3, dis_col, b3.reshape(1, _C), W3)
    a0, a1 = segsum(ys4)
    out4, xw5, ys5 = _tc_layer_mid(a0, a1, xw4, dis_col, b3.reshape(1, _C), W3)
    a0, a1 = segsum(ys5)
    out5 = _tc_layer_last(a0, a1, xw5, dis_col, b3.reshape(1, _C))

    l1s = jnp.stack([
        lin1W[:, 0 * _C:1 * _C].T, lin1W[:, 1 * _C:2 * _C].T,
        lin1W[:, 2 * _C:3 * _C].T, lin1W[:, 3 * _C:4 * _C].T,
        lin1W[:, 4 * _C:5 * _C].T, lin1W[:, 5 * _C:6 * _C].T,
    ])
    la_row = lin1W[:, 6 * _C].reshape(1, 32)
    act_col = action.reshape(_N, 1)
    y = _tc_readout(out1, out2, out3, out4, out5, state, act_col, l1s,
                    la_row, lin1b.reshape(1, 32), lin2W.T,
                    lin2b.reshape(1, 32), lin3W.T, lin3b.reshape(1, 1))
    return y.reshape(_N // 8)


# speedup vs baseline: 19.3045x; 1.4067x over previous
import functools

import jax
import jax.numpy as jnp
from jax import lax
from jax.experimental import pallas as pl
from jax.experimental.pallas import tpu as pltpu
from jax.experimental.pallas import tpu_sc as plsc

_N = 10000
_C = 128
_E = 320000
_NW = 32
_PER_W = _E // _NW
_CH = 80
_NCH = _PER_W // _CH
_RPT = 624

_mesh = plsc.VectorSubcoreMesh(core_axis_name="c", subcore_axis_name="s")


@functools.partial(
    pl.kernel,
    out_type=jax.ShapeDtypeStruct((2, _N), jnp.float32),
    scratch_types=[
        pltpu.VMEM_SHARED((_N,), jnp.float32),
        pltpu.VMEM((_CH,), jnp.float32),
        pltpu.VMEM((_CH,), jnp.int32),
        pltpu.VMEM((_CH,), jnp.int32),
        pltpu.SemaphoreType.DMA,
        pltpu.SemaphoreType.DMA,
    ],
    mesh=_mesh,
)
def _sc_degree(dst_hbm, zeros_hbm, ones_hbm, out_hbm, shared_deg, ones_v,
               didx_a, didx_b, sem_da, sem_db):
    cid = lax.axis_index("c")
    sid = lax.axis_index("s")
    base = (cid * 16 + sid) * _PER_W

    @pl.when(sid == 0)
    def _():
        pltpu.sync_copy(zeros_hbm, shared_deg)

    pltpu.sync_copy(ones_hbm, ones_v)
    plsc.subcore_barrier()

    def body(j, _):
        o0 = base + 2 * j * _CH
        h_a = pltpu.async_copy(dst_hbm.at[pl.ds(o0, _CH)], didx_a, sem_da)
        h_b = pltpu.async_copy(dst_hbm.at[pl.ds(o0 + _CH, _CH)], didx_b,
                               sem_db)
        h_a.wait()
        pltpu.sync_copy(ones_v, shared_deg.at[didx_a], add=True)
        h_b.wait()
        pltpu.sync_copy(ones_v, shared_deg.at[didx_b], add=True)
        return 0

    lax.fori_loop(0, _NCH // 2, body, 0)
    h_t = pltpu.async_copy(dst_hbm.at[pl.ds(base + (_NCH - 1) * _CH, _CH)],
                           didx_a, sem_da)
    h_t.wait()
    pltpu.sync_copy(ones_v, shared_deg.at[didx_a], add=True)
    plsc.subcore_barrier()

    @pl.when(sid == 0)
    def _():
        pltpu.sync_copy(shared_deg, out_hbm.at[cid])


@functools.partial(
    pl.kernel,
    out_type=jax.ShapeDtypeStruct((2, _N, _C), jnp.float32),
    scratch_types=[
        pltpu.VMEM_SHARED((_N, _C), jnp.float32),
        pltpu.VMEM((_CH,), jnp.int32),
        pltpu.VMEM((_CH,), jnp.int32),
        pltpu.VMEM((_CH,), jnp.int32),
        pltpu.VMEM((_CH,), jnp.int32),
        pltpu.VMEM((_CH, _C), jnp.float32),
        pltpu.VMEM((_CH, _C), jnp.float32),
        pltpu.SemaphoreType.DMA,
        pltpu.SemaphoreType.DMA,
        pltpu.SemaphoreType.DMA,
        pltpu.SemaphoreType.DMA,
        pltpu.SemaphoreType.DMA,
        pltpu.SemaphoreType.DMA,
    ],
    mesh=_mesh,
)
def _sc_segsum(ys_hbm, src_hbm, dst_hbm, zeros_hbm, out_hbm, shared_acc,
               sidx_a, sidx_b, didx_a, didx_b, rows_a, rows_b, sem_sa,
               sem_sb, sem_da, sem_db, sem_a, sem_b):
    cid = lax.axis_index("c")
    sid = lax.axis_index("s")
    w = cid * 16 + sid

    @pl.when(sid < 15)
    def _():
        pltpu.sync_copy(zeros_hbm.at[pl.ds(sid * _RPT, _RPT)],
                        shared_acc.at[pl.ds(sid * _RPT, _RPT)])

    @pl.when(sid == 15)
    def _():
        pltpu.sync_copy(zeros_hbm.at[pl.ds(15 * _RPT, _N - 15 * _RPT)],
                        shared_acc.at[pl.ds(15 * _RPT, _N - 15 * _RPT)])

    base = (cid * 16 + sid) * _PER_W
    plsc.subcore_barrier()

    def _issue_idx(i, sidx, sem_s, didx, sem_d):
        off = base + i * _CH
        pltpu.async_copy(src_hbm.at[pl.ds(off, _CH)], sidx, sem_s)
        pltpu.async_copy(dst_hbm.at[pl.ds(off, _CH)], didx, sem_d)

    def _wait(src_like, dst, sem):
        pltpu.make_async_copy(src_like, dst, sem).wait()

    _issue_idx(0, sidx_a, sem_sa, didx_a, sem_da)
    _issue_idx(1, sidx_b, sem_sb, didx_b, sem_db)
    _wait(src_hbm.at[pl.ds(0, _CH)], sidx_a, sem_sa)
    pltpu.async_copy(ys_hbm.at[sidx_a], rows_a, sem_a)
    _wait(src_hbm.at[pl.ds(0, _CH)], sidx_b, sem_sb)
    pltpu.async_copy(ys_hbm.at[sidx_b], rows_b, sem_b)

    def _step(i, sidx, sem_s, didx, sem_d, rows, sem_g):
        _wait(ys_hbm.at[pl.ds(0, _CH)], rows, sem_g)
        off2 = base + (i + 2) * _CH
        pltpu.async_copy(src_hbm.at[pl.ds(off2, _CH)], sidx, sem_s)
        _wait(src_hbm.at[pl.ds(0, _CH)], didx, sem_d)
        pltpu.sync_copy(rows, shared_acc.at[didx], add=True)
        pltpu.async_copy(dst_hbm.at[pl.ds(off2, _CH)], didx, sem_d)
        _wait(src_hbm.at[pl.ds(0, _CH)], sidx, sem_s)
        pltpu.async_copy(ys_hbm.at[sidx], rows, sem_g)

    def body(i, _):
        @pl.when(i % 2 == 0)
        def _():
            _step(i, sidx_a, sem_sa, didx_a, sem_da, rows_a, sem_a)

        @pl.when(i % 2 == 1)
        def _():
            _step(i, sidx_b, sem_sb, didx_b, sem_db, rows_b, sem_b)

        return 0

    lax.fori_loop(0, _NCH - 2, body, 0)

    def _drain(didx, sem_d, rows, sem_g):
        _wait(ys_hbm.at[pl.ds(0, _CH)], rows, sem_g)
        _wait(src_hbm.at[pl.ds(0, _CH)], didx, sem_d)
        pltpu.sync_copy(rows, shared_acc.at[didx], add=True)

    _drain(didx_b, sem_db, rows_b, sem_b)
    _drain(didx_a, sem_da, rows_a, sem_a)
    plsc.subcore_barrier()

    @pl.when(sid < 15)
    def _():
        pltpu.sync_copy(shared_acc.at[pl.ds(sid * _RPT, _RPT)],
                        out_hbm.at[cid, pl.ds(sid * _RPT, _RPT)])

    @pl.when(sid == 15)
    def _():
        pltpu.sync_copy(shared_acc.at[pl.ds(15 * _RPT, _N - 15 * _RPT)],
                        out_hbm.at[cid, pl.ds(15 * _RPT, _N - 15 * _RPT)])


def _dis_body(deg_ref, out_ref):
    d = deg_ref[0:1, :] + deg_ref[1:2, :] + 1.0
    out_ref[...] = lax.rsqrt(d)


def _tc_dis(deg2):
    return pl.pallas_call(
        _dis_body,
        out_shape=jax.ShapeDtypeStruct((1, _N), jnp.float32),
    )(deg2)


_R = 2000


def _project_body(h_ref, w_ref, dis_ref, xw_ref, ys_ref):
    xw = jnp.dot(h_ref[...], w_ref[...], preferred_element_type=jnp.float32)
    xw_ref[...] = xw
    ys_ref[...] = xw * dis_ref[...]


def _tc_project(h, w, dis_col):
    grid = (_N // _R,)
    return pl.pallas_call(
        _project_body,
        grid=grid,
        in_specs=[
            pl.BlockSpec((_R, _C), lambda i: (i, 0)),
            pl.BlockSpec((_C, _C), lambda i: (0, 0)),
            pl.BlockSpec((_R, 1), lambda i: (i, 0)),
        ],
        out_specs=[
            pl.BlockSpec((_R, _C), lambda i: (i, 0)),
            pl.BlockSpec((_R, _C), lambda i: (i, 0)),
        ],
        out_shape=[
            jax.ShapeDtypeStruct((_N, _C), jnp.float32),
            jax.ShapeDtypeStruct((_N, _C), jnp.float32),
        ],
    )(h, w, dis_col)


def _layer_mid_body(a0_ref, a1_ref, xw_ref, dis_ref, b_ref, w_ref,
                    out_ref, xwn_ref, ysn_ref):
    d = dis_ref[...]
    agg = (a0_ref[...] + a1_ref[...]) * d + xw_ref[...] * (d * d) + b_ref[...]
    out = jnp.maximum(agg, 0.0)
    out_ref[...] = out
    xwn = jnp.dot(out, w_ref[...], preferred_element_type=jnp.float32)
    xwn_ref[...] = xwn
    ysn_ref[...] = xwn * d


def _tc_layer_mid(a0, a1, xw, dis_col, b_row, w_next):
    grid = (_N // _R,)
    return pl.pallas_call(
        _layer_mid_body,
        grid=grid,
        in_specs=[
            pl.BlockSpec((_R, _C), lambda i: (i, 0)),
            pl.BlockSpec((_R, _C), lambda i: (i, 0)),
            pl.BlockSpec((_R, _C), lambda i: (i, 0)),
            pl.BlockSpec((_R, 1), lambda i: (i, 0)),
            pl.BlockSpec((1, _C), lambda i: (0, 0)),
            pl.BlockSpec((_C, _C), lambda i: (0, 0)),
        ],
        out_specs=[
            pl.BlockSpec((_R, _C), lambda i: (i, 0)),
            pl.BlockSpec((_R, _C), lambda i: (i, 0)),
            pl.BlockSpec((_R, _C), lambda i: (i, 0)),
        ],
        out_shape=[
            jax.ShapeDtypeStruct((_N, _C), jnp.float32),
            jax.ShapeDtypeStruct((_N, _C), jnp.float32),
            jax.ShapeDtypeStruct((_N, _C), jnp.float32),
        ],
    )(a0, a1, xw, dis_col, b_row, w_next)


def _layer_last_body(a0_ref, a1_ref, xw_ref, dis_ref, b_ref, out_ref):
    d = dis_ref[...]
    agg = (a0_ref[...] + a1_ref[...]) * d + xw_ref[...] * (d * d) + b_ref[...]
    out_ref[...] = jnp.maximum(agg, 0.0)


def _tc_layer_last(a0, a1, xw, dis_col, b_row):
    grid = (_N // _R,)
    return pl.pallas_call(
        _layer_last_body,
        grid=grid,
        in_specs=[
            pl.BlockSpec((_R, _C), lambda i: (i, 0)),
            pl.BlockSpec((_R, _C), lambda i: (i, 0)),
            pl.BlockSpec((_R, _C), lambda i: (i, 0)),
            pl.BlockSpec((_R, 1), lambda i: (i, 0)),
            pl.BlockSpec((1, _C), lambda i: (0, 0)),
        ],
        out_specs=pl.BlockSpec((_R, _C), lambda i: (i, 0)),
        out_shape=jax.ShapeDtypeStruct((_N, _C), jnp.float32),
    )(a0, a1, xw, dis_col, b_row)


_G = _R // 8


def _readout_body(o1_ref, o2_ref, o3_ref, o4_ref, o5_ref, st_ref, act_ref,
                  l1_ref, la_ref, l1b_ref, l2_ref, l2b_ref, l3_ref, l3b_ref,
                  out_ref):
    f32 = jnp.float32
    x1 = jnp.dot(o1_ref[...], l1_ref[0], preferred_element_type=f32)
    x1 += jnp.dot(o2_ref[...], l1_ref[1], preferred_element_type=f32)
    x1 += jnp.dot(o3_ref[...], l1_ref[2], preferred_element_type=f32)
    x1 += jnp.dot(o4_ref[...], l1_ref[3], preferred_element_type=f32)
    x1 += jnp.dot(o5_ref[...], l1_ref[4], preferred_element_type=f32)
    x1 += jnp.dot(st_ref[...], l1_ref[5], preferred_element_type=f32)
    x1 += act_ref[...] * la_ref[...] + l1b_ref[...]
    x1 = jnp.maximum(x1, 0.0)
    x2 = jnp.maximum(
        jnp.dot(x1, l2_ref[...], preferred_element_type=f32) + l2b_ref[...],
        0.0)
    s = jnp.dot(x2, l3_ref[...], preferred_element_type=f32)
    rows = lax.broadcasted_iota(jnp.int32, (_G, _R), 1)
    grp = lax.broadcasted_iota(jnp.int32, (_G, _R), 0)
    pmat = jnp.where(rows // 8 == grp, 1.0, 0.0).astype(f32)
    res = jnp.dot(pmat, s, preferred_element_type=f32) + l3b_ref[...]
    out_ref[...] = res[None]


def _tc_readout(o1, o2, o3, o4, o5, st, act_col, l1s, la_row, l1b_row, l2t,
                l2b_row, l3col, l3b11):
    grid = (_N // _R,)
    rc = pl.BlockSpec((_R, _C), lambda i: (i, 0))
    return pl.pallas_call(
        _readout_body,
        grid=grid,
        in_specs=[
            rc, rc, rc, rc, rc, rc,
            pl.BlockSpec((_R, 1), lambda i: (i, 0)),
            pl.BlockSpec((6, _C, 32), lambda i: (0, 0, 0)),
            pl.BlockSpec((1, 32), lambda i: (0, 0)),
            pl.BlockSpec((1, 32), lambda i: (0, 0)),
            pl.BlockSpec((32, 32), lambda i: (0, 0)),
            pl.BlockSpec((1, 32), lambda i: (0, 0)),
            pl.BlockSpec((32, 1), lambda i: (0, 0)),
            pl.BlockSpec((1, 1), lambda i: (0, 0)),
        ],
        out_specs=pl.BlockSpec((1, _G, 1), lambda i: (i, 0, 0)),
        out_shape=jax.ShapeDtypeStruct((_N // _R, _G, 1), jnp.float32),
    )(o1, o2, o3, o4, o5, st, act_col, l1s, la_row, l1b_row, l2t, l2b_row,
      l3col, l3b11)


def kernel(state, edge_index, action, W1, b1, W2, b2, W3, b3, lin1W, lin1b,
           lin2W, lin2b, lin3W, lin3b):
    src = edge_index[0]
    dst = edge_index[1]
    zeros_n = jnp.zeros((_N,), jnp.float32)
    zeros_nc = jnp.zeros((_N, _C), jnp.float32)
    ones_ch = jnp.ones((_CH,), jnp.float32)

    deg2 = _sc_degree(dst, zeros_n, ones_ch)
    dis_row = _tc_dis(deg2)
    dis_col = dis_row.reshape(_N, 1)

    def segsum(ys):
        acc = _sc_segsum(ys, src, dst, zeros_nc)
        return acc[0], acc[1]

    xw1, ys1 = _tc_project(state, W1, dis_col)
    a0, a1 = segsum(ys1)
    out1, xw2, ys2 = _tc_layer_mid(a0, a1, xw1, dis_col, b1.reshape(1, _C), W2)
    a0, a1 = segsum(ys2)
    out2, xw3, ys3 = _tc_layer_mid(a0, a1, xw2, dis_col, b2.reshape(1, _C), W3)
    a0, a1 = segsum(ys3)
    out3, xw4, ys4 = _tc_layer_mid(a0, a1, xw3, dis_col, b3.reshape(1, _C), W3)
    a0, a1 = segsum(ys4)
    out4, xw5, ys5 = _tc_layer_mid(a0, a1, xw4, dis_col, b3.reshape(1, _C), W3)
    a0, a1 = segsum(ys5)
    out5 = _tc_layer_last(a0, a1, xw5, dis_col, b3.reshape(1, _C))

    l1s = jnp.stack([
        lin1W[:, 0 * _C:1 * _C].T, lin1W[:, 1 * _C:2 * _C].T,
        lin1W[:, 2 * _C:3 * _C].T, lin1W[:, 3 * _C:4 * _C].T,
        lin1W[:, 4 * _C:5 * _C].T, lin1W[:, 5 * _C:6 * _C].T,
    ])
    la_row = lin1W[:, 6 * _C].reshape(1, 32)
    act_col = action.reshape(_N, 1)
    y = _tc_readout(out1, out2, out3, out4, out5, state, act_col, l1s,
                    la_row, lin1b.reshape(1, 32), lin2W.T,
                    lin2b.reshape(1, 32), lin3W.T, lin3b.reshape(1, 1))
    return y.reshape(_N // 8)


# speedup vs baseline: 21.4451x; 1.1109x over previous
import functools

import jax
import jax.numpy as jnp
from jax import lax
from jax.experimental import pallas as pl
from jax.experimental.pallas import tpu as pltpu
from jax.experimental.pallas import tpu_sc as plsc

_N = 10000
_C = 128
_E = 320000
_NW = 32
_PER_W = _E // _NW
_CH = 128
_NCH = _PER_W // _CH
_TAIL = _PER_W - _NCH * _CH
_RPT = 624

_mesh = plsc.VectorSubcoreMesh(core_axis_name="c", subcore_axis_name="s")


@functools.partial(
    pl.kernel,
    out_type=jax.ShapeDtypeStruct((2, _N), jnp.float32),
    scratch_types=[
        pltpu.VMEM_SHARED((_N,), jnp.float32),
        pltpu.VMEM((_CH,), jnp.float32),
        pltpu.VMEM((_CH,), jnp.int32),
        pltpu.VMEM((_CH,), jnp.int32),
        pltpu.VMEM((_TAIL,), jnp.int32),
        pltpu.SemaphoreType.DMA,
        pltpu.SemaphoreType.DMA,
    ],
    mesh=_mesh,
)
def _sc_degree(dst_hbm, zeros_hbm, ones_hbm, out_hbm, shared_deg, ones_v,
               didx_a, didx_b, didx_t, sem_da, sem_db):
    cid = lax.axis_index("c")
    sid = lax.axis_index("s")
    base = (cid * 16 + sid) * _PER_W

    @pl.when(sid == 0)
    def _():
        pltpu.sync_copy(zeros_hbm, shared_deg)

    pltpu.sync_copy(ones_hbm, ones_v)
    plsc.subcore_barrier()

    def body(j, _):
        o0 = base + 2 * j * _CH
        h_a = pltpu.async_copy(dst_hbm.at[pl.ds(o0, _CH)], didx_a, sem_da)
        h_b = pltpu.async_copy(dst_hbm.at[pl.ds(o0 + _CH, _CH)], didx_b,
                               sem_db)
        h_a.wait()
        pltpu.sync_copy(ones_v, shared_deg.at[didx_a], add=True)
        h_b.wait()
        pltpu.sync_copy(ones_v, shared_deg.at[didx_b], add=True)
        return 0

    lax.fori_loop(0, _NCH // 2, body, 0)
    h_t = pltpu.async_copy(dst_hbm.at[pl.ds(base + _NCH * _CH, _TAIL)],
                           didx_t, sem_da)
    h_t.wait()
    pltpu.sync_copy(ones_v.at[pl.ds(0, _TAIL)], shared_deg.at[didx_t],
                    add=True)
    plsc.subcore_barrier()

    @pl.when(sid == 0)
    def _():
        pltpu.sync_copy(shared_deg, out_hbm.at[cid])


@functools.partial(
    pl.kernel,
    out_type=jax.ShapeDtypeStruct((2, _N, _C), jnp.float32),
    scratch_types=[
        pltpu.VMEM_SHARED((_N, _C), jnp.float32),
        pltpu.VMEM((_CH,), jnp.int32),
        pltpu.VMEM((_CH,), jnp.int32),
        pltpu.VMEM((_CH,), jnp.int32),
        pltpu.VMEM((_CH,), jnp.int32),
        pltpu.VMEM((_TAIL,), jnp.int32),
        pltpu.VMEM((_TAIL,), jnp.int32),
        pltpu.VMEM((_CH, _C), jnp.float32),
        pltpu.VMEM((_CH, _C), jnp.float32),
        pltpu.SemaphoreType.DMA,
        pltpu.SemaphoreType.DMA,
        pltpu.SemaphoreType.DMA,
        pltpu.SemaphoreType.DMA,
        pltpu.SemaphoreType.DMA,
        pltpu.SemaphoreType.DMA,
    ],
    mesh=_mesh,
)
def _sc_segsum(ys_hbm, src_hbm, dst_hbm, zeros_hbm, out_hbm, shared_acc,
               sidx_a, sidx_b, didx_a, didx_b, sidx_t, didx_t, rows_a,
               rows_b, sem_sa, sem_sb, sem_da, sem_db, sem_a, sem_b):
    cid = lax.axis_index("c")
    sid = lax.axis_index("s")
    w = cid * 16 + sid

    @pl.when(sid < 15)
    def _():
        pltpu.sync_copy(zeros_hbm.at[pl.ds(sid * _RPT, _RPT)],
                        shared_acc.at[pl.ds(sid * _RPT, _RPT)])

    @pl.when(sid == 15)
    def _():
        pltpu.sync_copy(zeros_hbm.at[pl.ds(15 * _RPT, _N - 15 * _RPT)],
                        shared_acc.at[pl.ds(15 * _RPT, _N - 15 * _RPT)])

    base = (cid * 16 + sid) * _PER_W
    plsc.subcore_barrier()

    def _issue_idx(i, sidx, sem_s, didx, sem_d):
        off = base + i * _CH
        pltpu.async_copy(src_hbm.at[pl.ds(off, _CH)], sidx, sem_s)
        pltpu.async_copy(dst_hbm.at[pl.ds(off, _CH)], didx, sem_d)

    def _wait(src_like, dst, sem):
        pltpu.make_async_copy(src_like, dst, sem).wait()

    _issue_idx(0, sidx_a, sem_sa, didx_a, sem_da)
    _issue_idx(1, sidx_b, sem_sb, didx_b, sem_db)
    _wait(src_hbm.at[pl.ds(0, _CH)], sidx_a, sem_sa)
    pltpu.async_copy(ys_hbm.at[sidx_a], rows_a, sem_a)
    _wait(src_hbm.at[pl.ds(0, _CH)], sidx_b, sem_sb)
    pltpu.async_copy(ys_hbm.at[sidx_b], rows_b, sem_b)

    def _step(i, sidx, sem_s, didx, sem_d, rows, sem_g):
        _wait(ys_hbm.at[pl.ds(0, _CH)], rows, sem_g)
        off2 = base + (i + 2) * _CH
        pltpu.async_copy(src_hbm.at[pl.ds(off2, _CH)], sidx, sem_s)
        _wait(src_hbm.at[pl.ds(0, _CH)], didx, sem_d)
        pltpu.sync_copy(rows, shared_acc.at[didx], add=True)
        pltpu.async_copy(dst_hbm.at[pl.ds(off2, _CH)], didx, sem_d)
        _wait(src_hbm.at[pl.ds(0, _CH)], sidx, sem_s)
        pltpu.async_copy(ys_hbm.at[sidx], rows, sem_g)

    def body(i, _):
        @pl.when(i % 2 == 0)
        def _():
            _step(i, sidx_a, sem_sa, didx_a, sem_da, rows_a, sem_a)

        @pl.when(i % 2 == 1)
        def _():
            _step(i, sidx_b, sem_sb, didx_b, sem_db, rows_b, sem_b)

        return 0

    lax.fori_loop(0, _NCH - 2, body, 0)

    def _drain(didx, sem_d, rows, sem_g):
        _wait(ys_hbm.at[pl.ds(0, _CH)], rows, sem_g)
        _wait(src_hbm.at[pl.ds(0, _CH)], didx, sem_d)
        pltpu.sync_copy(rows, shared_acc.at[didx], add=True)

    _drain(didx_a, sem_da, rows_a, sem_a)
    _drain(didx_b, sem_db, rows_b, sem_b)

    o_t = base + _NCH * _CH
    pltpu.async_copy(src_hbm.at[pl.ds(o_t, _TAIL)], sidx_t, sem_sa)
    pltpu.async_copy(dst_hbm.at[pl.ds(o_t, _TAIL)], didx_t, sem_da)
    _wait(src_hbm.at[pl.ds(0, _TAIL)], sidx_t, sem_sa)
    pltpu.async_copy(ys_hbm.at[sidx_t], rows_a.at[pl.ds(0, _TAIL)],
                     sem_a).wait()
    _wait(dst_hbm.at[pl.ds(0, _TAIL)], didx_t, sem_da)
    pltpu.sync_copy(rows_a.at[pl.ds(0, _TAIL)], shared_acc.at[didx_t],
                    add=True)
    plsc.subcore_barrier()

    @pl.when(sid < 15)
    def _():
        pltpu.sync_copy(shared_acc.at[pl.ds(sid * _RPT, _RPT)],
                        out_hbm.at[cid, pl.ds(sid * _RPT, _RPT)])

    @pl.when(sid == 15)
    def _():
        pltpu.sync_copy(shared_acc.at[pl.ds(15 * _RPT, _N - 15 * _RPT)],
                        out_hbm.at[cid, pl.ds(15 * _RPT, _N - 15 * _RPT)])


def _dis_body(deg_ref, out_ref):
    d = deg_ref[0:1, :] + deg_ref[1:2, :] + 1.0
    out_ref[...] = lax.rsqrt(d)


def _tc_dis(deg2):
    return pl.pallas_call(
        _dis_body,
        out_shape=jax.ShapeDtypeStruct((1, _N), jnp.float32),
    )(deg2)


_R = 2000


def _project_body(h_ref, w_ref, dis_ref, xw_ref, ys_ref):
    xw = jnp.dot(h_ref[...], w_ref[...], preferred_element_type=jnp.float32)
    xw_ref[...] = xw
    ys_ref[...] = xw * dis_ref[...]


def _tc_project(h, w, dis_col):
    grid = (_N // _R,)
    return pl.pallas_call(
        _project_body,
        grid=grid,
        in_specs=[
            pl.BlockSpec((_R, _C), lambda i: (i, 0)),
            pl.BlockSpec((_C, _C), lambda i: (0, 0)),
            pl.BlockSpec((_R, 1), lambda i: (i, 0)),
        ],
        out_specs=[
            pl.BlockSpec((_R, _C), lambda i: (i, 0)),
            pl.BlockSpec((_R, _C), lambda i: (i, 0)),
        ],
        out_shape=[
            jax.ShapeDtypeStruct((_N, _C), jnp.float32),
            jax.ShapeDtypeStruct((_N, _C), jnp.float32),
        ],
    )(h, w, dis_col)


def _layer_mid_body(a0_ref, a1_ref, xw_ref, dis_ref, b_ref, w_ref,
                    out_ref, xwn_ref, ysn_ref):
    d = dis_ref[...]
    agg = (a0_ref[...] + a1_ref[...]) * d + xw_ref[...] * (d * d) + b_ref[...]
    out = jnp.maximum(agg, 0.0)
    out_ref[...] = out
    xwn = jnp.dot(out, w_ref[...], preferred_element_type=jnp.float32)
    xwn_ref[...] = xwn
    ysn_ref[...] = xwn * d


def _tc_layer_mid(a0, a1, xw, dis_col, b_row, w_next):
    grid = (_N // _R,)
    return pl.pallas_call(
        _layer_mid_body,
        grid=grid,
        in_specs=[
            pl.BlockSpec((_R, _C), lambda i: (i, 0)),
            pl.BlockSpec((_R, _C), lambda i: (i, 0)),
            pl.BlockSpec((_R, _C), lambda i: (i, 0)),
            pl.BlockSpec((_R, 1), lambda i: (i, 0)),
            pl.BlockSpec((1, _C), lambda i: (0, 0)),
            pl.BlockSpec((_C, _C), lambda i: (0, 0)),
        ],
        out_specs=[
            pl.BlockSpec((_R, _C), lambda i: (i, 0)),
            pl.BlockSpec((_R, _C), lambda i: (i, 0)),
            pl.BlockSpec((_R, _C), lambda i: (i, 0)),
        ],
        out_shape=[
            jax.ShapeDtypeStruct((_N, _C), jnp.float32),
            jax.ShapeDtypeStruct((_N, _C), jnp.float32),
            jax.ShapeDtypeStruct((_N, _C), jnp.float32),
        ],
    )(a0, a1, xw, dis_col, b_row, w_next)


def _layer_last_body(a0_ref, a1_ref, xw_ref, dis_ref, b_ref, out_ref):
    d = dis_ref[...]
    agg = (a0_ref[...] + a1_ref[...]) * d + xw_ref[...] * (d * d) + b_ref[...]
    out_ref[...] = jnp.maximum(agg, 0.0)


def _tc_layer_last(a0, a1, xw, dis_col, b_row):
    grid = (_N // _R,)
    return pl.pallas_call(
        _layer_last_body,
        grid=grid,
        in_specs=[
            pl.BlockSpec((_R, _C), lambda i: (i, 0)),
            pl.BlockSpec((_R, _C), lambda i: (i, 0)),
            pl.BlockSpec((_R, _C), lambda i: (i, 0)),
            pl.BlockSpec((_R, 1), lambda i: (i, 0)),
            pl.BlockSpec((1, _C), lambda i: (0, 0)),
        ],
        out_specs=pl.BlockSpec((_R, _C), lambda i: (i, 0)),
        out_shape=jax.ShapeDtypeStruct((_N, _C), jnp.float32),
    )(a0, a1, xw, dis_col, b_row)


_G = _R // 8


def _readout_body(o1_ref, o2_ref, o3_ref, o4_ref, o5_ref, st_ref, act_ref,
                  l1_ref, la_ref, l1b_ref, l2_ref, l2b_ref, l3_ref, l3b_ref,
                  out_ref):
    f32 = jnp.float32
    x1 = jnp.dot(o1_ref[...], l1_ref[0], preferred_element_type=f32)
    x1 += jnp.dot(o2_ref[...], l1_ref[1], preferred_element_type=f32)
    x1 += jnp.dot(o3_ref[...], l1_ref[2], preferred_element_type=f32)
    x1 += jnp.dot(o4_ref[...], l1_ref[3], preferred_element_type=f32)
    x1 += jnp.dot(o5_ref[...], l1_ref[4], preferred_element_type=f32)
    x1 += jnp.dot(st_ref[...], l1_ref[5], preferred_element_type=f32)
    x1 += act_ref[...] * la_ref[...] + l1b_ref[...]
    x1 = jnp.maximum(x1, 0.0)
    x2 = jnp.maximum(
        jnp.dot(x1, l2_ref[...], preferred_element_type=f32) + l2b_ref[...],
        0.0)
    s = jnp.dot(x2, l3_ref[...], preferred_element_type=f32)
    rows = lax.broadcasted_iota(jnp.int32, (_G, _R), 1)
    grp = lax.broadcasted_iota(jnp.int32, (_G, _R), 0)
    pmat = jnp.where(rows // 8 == grp, 1.0, 0.0).astype(f32)
    res = jnp.dot(pmat, s, preferred_element_type=f32) + l3b_ref[...]
    out_ref[...] = res[None]


def _tc_readout(o1, o2, o3, o4, o5, st, act_col, l1s, la_row, l1b_row, l2t,
                l2b_row, l3col, l3b11):
    grid = (_N // _R,)
    rc = pl.BlockSpec((_R, _C), lambda i: (i, 0))
    return pl.pallas_call(
        _readout_body,
        grid=grid,
        in_specs=[
            rc, rc, rc, rc, rc, rc,
            pl.BlockSpec((_R, 1), lambda i: (i, 0)),
            pl.BlockSpec((6, _C, 32), lambda i: (0, 0, 0)),
            pl.BlockSpec((1, 32), lambda i: (0, 0)),
            pl.BlockSpec((1, 32), lambda i: (0, 0)),
            pl.BlockSpec((32, 32), lambda i: (0, 0)),
            pl.BlockSpec((1, 32), lambda i: (0, 0)),
            pl.BlockSpec((32, 1), lambda i: (0, 0)),
            pl.BlockSpec((1, 1), lambda i: (0, 0)),
        ],
        out_specs=pl.BlockSpec((1, _G, 1), lambda i: (i, 0, 0)),
        out_shape=jax.ShapeDtypeStruct((_N // _R, _G, 1), jnp.float32),
    )(o1, o2, o3, o4, o5, st, act_col, l1s, la_row, l1b_row, l2t, l2b_row,
      l3col, l3b11)


def kernel(state, edge_index, action, W1, b1, W2, b2, W3, b3, lin1W, lin1b,
           lin2W, lin2b, lin3W, lin3b):
    src = edge_index[0]
    dst = edge_index[1]
    zeros_n = jnp.zeros((_N,), jnp.float32)
    zeros_nc = jnp.zeros((_N, _C), jnp.float32)
    ones_ch = jnp.ones((_CH,), jnp.float32)

    deg2 = _sc_degree(dst, zeros_n, ones_ch)
    dis_row = _tc_dis(deg2)
    dis_col = dis_row.reshape(_N, 1)

    def segsum(ys):
        acc = _sc_segsum(ys, src, dst, zeros_nc)
        return acc[0], acc[1]

    xw1, ys1 = _tc_project(state, W1, dis_col)
    a0, a1 = segsum(ys1)
    out1, xw2, ys2 = _tc_layer_mid(a0, a1, xw1, dis_col, b1.reshape(1, _C), W2)
    a0, a1 = segsum(ys2)
    out2, xw3, ys3 = _tc_layer_mid(a0, a1, xw2, dis_col, b2.reshape(1, _C), W3)
    a0, a1 = segsum(ys3)
    out3, xw4, ys4 = _tc_layer_mid(a0, a1, xw3, dis_col, b3.reshape(1, _C), W3)
    a0, a1 = segsum(ys4)
    out4, xw5, ys5 = _tc_layer_mid(a0, a1, xw4, dis_col, b3.reshape(1, _C), W3)
    a0, a1 = segsum(ys5)
    out5 = _tc_layer_last(a0, a1, xw5, dis_col, b3.reshape(1, _C))

    l1s = jnp.stack([
        lin1W[:, 0 * _C:1 * _C].T, lin1W[:, 1 * _C:2 * _C].T,
        lin1W[:, 2 * _C:3 * _C].T, lin1W[:, 3 * _C:4 * _C].T,
        lin1W[:, 4 * _C:5 * _C].T, lin1W[:, 5 * _C:6 * _C].T,
    ])
    la_row = lin1W[:, 6 * _C].reshape(1, 32)
    act_col = action.reshape(_N, 1)
    y = _tc_readout(out1, out2, out3, out4, out5, state, act_col, l1s,
                    la_row, lin1b.reshape(1, 32), lin2W.T,
                    lin2b.reshape(1, 32), lin3W.T, lin3b.reshape(1, 1))
    return y.reshape(_N // 8)


# speedup vs baseline: 22.0705x; 1.0292x over previous
import functools

import jax
import jax.numpy as jnp
from jax import lax
from jax.experimental import pallas as pl
from jax.experimental.pallas import tpu as pltpu
from jax.experimental.pallas import tpu_sc as plsc

_N = 10000
_C = 128
_E = 320000
_NW = 32
_PER_W = _E // _NW
_CH = 128
_NCH = _PER_W // _CH
_TAIL = _PER_W - _NCH * _CH
_RPT = 624

_mesh = plsc.VectorSubcoreMesh(core_axis_name="c", subcore_axis_name="s")


_RING = 13


@functools.partial(
    pl.kernel,
    out_type=jax.ShapeDtypeStruct((2, _N), jnp.float32),
    scratch_types=[
        pltpu.VMEM_SHARED((_N,), jnp.float32),
        pltpu.VMEM((_CH,), jnp.float32),
        pltpu.VMEM((_NCH, 1, _CH), jnp.int32),
        pltpu.VMEM((1, _TAIL), jnp.int32),
        pltpu.SemaphoreType.DMA,
    ],
    mesh=_mesh,
)
def _sc_degree(dstf_hbm, dstt_hbm, zeros_hbm, ones_hbm, out_hbm, shared_deg,
               ones_v, didx3d, didx_t, sem_sc):
    cid = lax.axis_index("c")
    sid = lax.axis_index("s")
    w = cid * 16 + sid

    @pl.when(sid == 0)
    def _():
        pltpu.sync_copy(zeros_hbm, shared_deg)

    pltpu.sync_copy(ones_hbm, ones_v)
    pltpu.sync_copy(dstf_hbm.at[w], didx3d)
    pltpu.sync_copy(dstt_hbm.at[w], didx_t)
    plsc.subcore_barrier()

    def _wait_one():
        pltpu.make_async_copy(ones_v, shared_deg.at[didx3d.at[0, 0]],
                              sem_sc).wait()

    def ring(g, _):
        for k in range(_RING):
            pltpu.async_copy(ones_v, shared_deg.at[didx3d.at[g * _RING + k, 0]],
                             sem_sc, add=True)

        @pl.when(g > 0)
        def _():
            for k in range(_RING):
                _wait_one()

        return 0

    lax.fori_loop(0, _NCH // _RING, ring, 0)
    for k in range(_RING):
        _wait_one()
    pltpu.sync_copy(ones_v.at[pl.ds(0, _TAIL)], shared_deg.at[didx_t.at[0]],
                    add=True)
    plsc.subcore_barrier()

    @pl.when(sid == 0)
    def _():
        pltpu.sync_copy(shared_deg, out_hbm.at[cid])


@functools.partial(
    pl.kernel,
    out_type=jax.ShapeDtypeStruct((2, _N, _C), jnp.float32),
    scratch_types=[
        pltpu.VMEM_SHARED((_N, _C), jnp.float32),
        pltpu.VMEM((_CH,), jnp.int32),
        pltpu.VMEM((_CH,), jnp.int32),
        pltpu.VMEM((_CH,), jnp.int32),
        pltpu.VMEM((_CH,), jnp.int32),
        pltpu.VMEM((_TAIL,), jnp.int32),
        pltpu.VMEM((_TAIL,), jnp.int32),
        pltpu.VMEM((_CH, _C), jnp.float32),
        pltpu.VMEM((_CH, _C), jnp.float32),
        pltpu.SemaphoreType.DMA,
        pltpu.SemaphoreType.DMA,
        pltpu.SemaphoreType.DMA,
        pltpu.SemaphoreType.DMA,
        pltpu.SemaphoreType.DMA,
        pltpu.SemaphoreType.DMA,
    ],
    mesh=_mesh,
)
def _sc_segsum(ys_hbm, src_hbm, dst_hbm, zeros_hbm, out_hbm, shared_acc,
               sidx_a, sidx_b, didx_a, didx_b, sidx_t, didx_t, rows_a,
               rows_b, sem_sa, sem_sb, sem_da, sem_db, sem_a, sem_b):
    cid = lax.axis_index("c")
    sid = lax.axis_index("s")
    w = cid * 16 + sid

    @pl.when(sid < 15)
    def _():
        pltpu.sync_copy(zeros_hbm.at[pl.ds(sid * _RPT, _RPT)],
                        shared_acc.at[pl.ds(sid * _RPT, _RPT)])

    @pl.when(sid == 15)
    def _():
        pltpu.sync_copy(zeros_hbm.at[pl.ds(15 * _RPT, _N - 15 * _RPT)],
                        shared_acc.at[pl.ds(15 * _RPT, _N - 15 * _RPT)])

    base = (cid * 16 + sid) * _PER_W
    plsc.subcore_barrier()

    def _issue_idx(i, sidx, sem_s, didx, sem_d):
        off = base + i * _CH
        pltpu.async_copy(src_hbm.at[pl.ds(off, _CH)], sidx, sem_s)
        pltpu.async_copy(dst_hbm.at[pl.ds(off, _CH)], didx, sem_d)

    def _wait(src_like, dst, sem):
        pltpu.make_async_copy(src_like, dst, sem).wait()

    _issue_idx(0, sidx_a, sem_sa, didx_a, sem_da)
    _issue_idx(1, sidx_b, sem_sb, didx_b, sem_db)
    _wait(src_hbm.at[pl.ds(0, _CH)], sidx_a, sem_sa)
    pltpu.async_copy(ys_hbm.at[sidx_a], rows_a, sem_a)
    _wait(src_hbm.at[pl.ds(0, _CH)], sidx_b, sem_sb)
    pltpu.async_copy(ys_hbm.at[sidx_b], rows_b, sem_b)

    def _step(i, sidx, sem_s, didx, sem_d, rows, sem_g):
        _wait(ys_hbm.at[pl.ds(0, _CH)], rows, sem_g)
        off2 = base + (i + 2) * _CH
        pltpu.async_copy(src_hbm.at[pl.ds(off2, _CH)], sidx, sem_s)
        _wait(src_hbm.at[pl.ds(0, _CH)], didx, sem_d)
        pltpu.sync_copy(rows, shared_acc.at[didx], add=True)
        pltpu.async_copy(dst_hbm.at[pl.ds(off2, _CH)], didx, sem_d)
        _wait(src_hbm.at[pl.ds(0, _CH)], sidx, sem_s)
        pltpu.async_copy(ys_hbm.at[sidx], rows, sem_g)

    def body(i, _):
        @pl.when(i % 2 == 0)
        def _():
            _step(i, sidx_a, sem_sa, didx_a, sem_da, rows_a, sem_a)

        @pl.when(i % 2 == 1)
        def _():
            _step(i, sidx_b, sem_sb, didx_b, sem_db, rows_b, sem_b)

        return 0

    lax.fori_loop(0, _NCH - 2, body, 0)

    def _drain(didx, sem_d, rows, sem_g):
        _wait(ys_hbm.at[pl.ds(0, _CH)], rows, sem_g)
        _wait(src_hbm.at[pl.ds(0, _CH)], didx, sem_d)
        pltpu.sync_copy(rows, shared_acc.at[didx], add=True)

    _drain(didx_a, sem_da, rows_a, sem_a)
    _drain(didx_b, sem_db, rows_b, sem_b)

    o_t = base + _NCH * _CH
    pltpu.async_copy(src_hbm.at[pl.ds(o_t, _TAIL)], sidx_t, sem_sa)
    pltpu.async_copy(dst_hbm.at[pl.ds(o_t, _TAIL)], didx_t, sem_da)
    _wait(src_hbm.at[pl.ds(0, _TAIL)], sidx_t, sem_sa)
    pltpu.async_copy(ys_hbm.at[sidx_t], rows_a.at[pl.ds(0, _TAIL)],
                     sem_a).wait()
    _wait(dst_hbm.at[pl.ds(0, _TAIL)], didx_t, sem_da)
    pltpu.sync_copy(rows_a.at[pl.ds(0, _TAIL)], shared_acc.at[didx_t],
                    add=True)
    plsc.subcore_barrier()

    @pl.when(sid < 15)
    def _():
        pltpu.sync_copy(shared_acc.at[pl.ds(sid * _RPT, _RPT)],
                        out_hbm.at[cid, pl.ds(sid * _RPT, _RPT)])

    @pl.when(sid == 15)
    def _():
        pltpu.sync_copy(shared_acc.at[pl.ds(15 * _RPT, _N - 15 * _RPT)],
                        out_hbm.at[cid, pl.ds(15 * _RPT, _N - 15 * _RPT)])


def _dis_body(deg_ref, out_ref):
    d = deg_ref[0:1, :] + deg_ref[1:2, :] + 1.0
    out_ref[...] = lax.rsqrt(d)


def _tc_dis(deg2):
    return pl.pallas_call(
        _dis_body,
        out_shape=jax.ShapeDtypeStruct((1, _N), jnp.float32),
    )(deg2)


_R = 2000


def _matmul_body(h_ref, w_ref, xw_ref):
    xw_ref[...] = jnp.dot(h_ref[...], w_ref[...],
                          preferred_element_type=jnp.float32)


def _tc_matmul(h, w):
    grid = (_N // _R,)
    return pl.pallas_call(
        _matmul_body,
        grid=grid,
        in_specs=[
            pl.BlockSpec((_R, _C), lambda i: (i, 0)),
            pl.BlockSpec((_C, _C), lambda i: (0, 0)),
        ],
        out_specs=pl.BlockSpec((_R, _C), lambda i: (i, 0)),
        out_shape=jax.ShapeDtypeStruct((_N, _C), jnp.float32),
    )(h, w)


def _scale_body(xw_ref, dis_ref, ys_ref):
    ys_ref[...] = xw_ref[...] * dis_ref[...]


def _tc_scale(xw, dis_col):
    grid = (_N // _R,)
    return pl.pallas_call(
        _scale_body,
        grid=grid,
        in_specs=[
            pl.BlockSpec((_R, _C), lambda i: (i, 0)),
            pl.BlockSpec((_R, 1), lambda i: (i, 0)),
        ],
        out_specs=pl.BlockSpec((_R, _C), lambda i: (i, 0)),
        out_shape=jax.ShapeDtypeStruct((_N, _C), jnp.float32),
    )(xw, dis_col)


def _layer_mid_body(a0_ref, a1_ref, xw_ref, dis_ref, b_ref, w_ref,
                    out_ref, xwn_ref, ysn_ref):
    d = dis_ref[...]
    agg = (a0_ref[...] + a1_ref[...]) * d + xw_ref[...] * (d * d) + b_ref[...]
    out = jnp.maximum(agg, 0.0)
    out_ref[...] = out
    xwn = jnp.dot(out, w_ref[...], preferred_element_type=jnp.float32)
    xwn_ref[...] = xwn
    ysn_ref[...] = xwn * d


def _tc_layer_mid(a0, a1, xw, dis_col, b_row, w_next):
    grid = (_N // _R,)
    return pl.pallas_call(
        _layer_mid_body,
        grid=grid,
        in_specs=[
            pl.BlockSpec((_R, _C), lambda i: (i, 0)),
            pl.BlockSpec((_R, _C), lambda i: (i, 0)),
            pl.BlockSpec((_R, _C), lambda i: (i, 0)),
            pl.BlockSpec((_R, 1), lambda i: (i, 0)),
            pl.BlockSpec((1, _C), lambda i: (0, 0)),
            pl.BlockSpec((_C, _C), lambda i: (0, 0)),
        ],
        out_specs=[
            pl.BlockSpec((_R, _C), lambda i: (i, 0)),
            pl.BlockSpec((_R, _C), lambda i: (i, 0)),
            pl.BlockSpec((_R, _C), lambda i: (i, 0)),
        ],
        out_shape=[
            jax.ShapeDtypeStruct((_N, _C), jnp.float32),
            jax.ShapeDtypeStruct((_N, _C), jnp.float32),
            jax.ShapeDtypeStruct((_N, _C), jnp.float32),
        ],
    )(a0, a1, xw, dis_col, b_row, w_next)


def _layer_last_body(a0_ref, a1_ref, xw_ref, dis_ref, b_ref, out_ref):
    d = dis_ref[...]
    agg = (a0_ref[...] + a1_ref[...]) * d + xw_ref[...] * (d * d) + b_ref[...]
    out_ref[...] = jnp.maximum(agg, 0.0)


def _tc_layer_last(a0, a1, xw, dis_col, b_row):
    grid = (_N // _R,)
    return pl.pallas_call(
        _layer_last_body,
        grid=grid,
        in_specs=[
            pl.BlockSpec((_R, _C), lambda i: (i, 0)),
            pl.BlockSpec((_R, _C), lambda i: (i, 0)),
            pl.BlockSpec((_R, _C), lambda i: (i, 0)),
            pl.BlockSpec((_R, 1), lambda i: (i, 0)),
            pl.BlockSpec((1, _C), lambda i: (0, 0)),
        ],
        out_specs=pl.BlockSpec((_R, _C), lambda i: (i, 0)),
        out_shape=jax.ShapeDtypeStruct((_N, _C), jnp.float32),
    )(a0, a1, xw, dis_col, b_row)


_G = _R // 8


def _readout_body(o1_ref, o2_ref, o3_ref, o4_ref, o5_ref, st_ref, act_ref,
                  l1_ref, la_ref, l1b_ref, l2_ref, l2b_ref, l3_ref, l3b_ref,
                  out_ref):
    f32 = jnp.float32
    x1 = jnp.dot(o1_ref[...], l1_ref[0], preferred_element_type=f32)
    x1 += jnp.dot(o2_ref[...], l1_ref[1], preferred_element_type=f32)
    x1 += jnp.dot(o3_ref[...], l1_ref[2], preferred_element_type=f32)
    x1 += jnp.dot(o4_ref[...], l1_ref[3], preferred_element_type=f32)
    x1 += jnp.dot(o5_ref[...], l1_ref[4], preferred_element_type=f32)
    x1 += jnp.dot(st_ref[...], l1_ref[5], preferred_element_type=f32)
    x1 += act_ref[...] * la_ref[...] + l1b_ref[...]
    x1 = jnp.maximum(x1, 0.0)
    x2 = jnp.maximum(
        jnp.dot(x1, l2_ref[...], preferred_element_type=f32) + l2b_ref[...],
        0.0)
    s = jnp.dot(x2, l3_ref[...], preferred_element_type=f32)
    rows = lax.broadcasted_iota(jnp.int32, (_G, _R), 1)
    grp = lax.broadcasted_iota(jnp.int32, (_G, _R), 0)
    pmat = jnp.where(rows // 8 == grp, 1.0, 0.0).astype(f32)
    res = jnp.dot(pmat, s, preferred_element_type=f32) + l3b_ref[...]
    out_ref[...] = res[None]


def _tc_readout(o1, o2, o3, o4, o5, st, act_col, l1s, la_row, l1b_row, l2t,
                l2b_row, l3col, l3b11):
    grid = (_N // _R,)
    rc = pl.BlockSpec((_R, _C), lambda i: (i, 0))
    return pl.pallas_call(
        _readout_body,
        grid=grid,
        in_specs=[
            rc, rc, rc, rc, rc, rc,
            pl.BlockSpec((_R, 1), lambda i: (i, 0)),
            pl.BlockSpec((6, _C, 32), lambda i: (0, 0, 0)),
            pl.BlockSpec((1, 32), lambda i: (0, 0)),
            pl.BlockSpec((1, 32), lambda i: (0, 0)),
            pl.BlockSpec((32, 32), lambda i: (0, 0)),
            pl.BlockSpec((1, 32), lambda i: (0, 0)),
            pl.BlockSpec((32, 1), lambda i: (0, 0)),
            pl.BlockSpec((1, 1), lambda i: (0, 0)),
        ],
        out_specs=pl.BlockSpec((1, _G, 1), lambda i: (i, 0, 0)),
        out_shape=jax.ShapeDtypeStruct((_N // _R, _G, 1), jnp.float32),
    )(o1, o2, o3, o4, o5, st, act_col, l1s, la_row, l1b_row, l2t, l2b_row,
      l3col, l3b11)


def kernel(state, edge_index, action, W1, b1, W2, b2, W3, b3, lin1W, lin1b,
           lin2W, lin2b, lin3W, lin3b):
    src = edge_index[0]
    dst = edge_index[1]
    dstw = dst.reshape(_NW, _PER_W)
    dstf = dstw[:, :_NCH * _CH].reshape(_NW, _NCH, 1, _CH)
    dstt = dstw[:, _NCH * _CH:].reshape(_NW, 1, _TAIL)
    zeros_n = jnp.zeros((_N,), jnp.float32)
    zeros_nc = jnp.zeros((_N, _C), jnp.float32)
    ones_ch = jnp.ones((_CH,), jnp.float32)

    xw1 = _tc_matmul(state, W1)
    deg2 = _sc_degree(dstf, dstt, zeros_n, ones_ch)
    dis_row = _tc_dis(deg2)
    dis_col = dis_row.reshape(_N, 1)

    def segsum(ys):
        acc = _sc_segsum(ys, src, dst, zeros_nc)
        return acc[0], acc[1]

    ys1 = _tc_scale(xw1, dis_col)
    a0, a1 = segsum(ys1)
    out1, xw2, ys2 = _tc_layer_mid(a0, a1, xw1, dis_col, b1.reshape(1, _C), W2)
    a0, a1 = segsum(ys2)
    out2, xw3, ys3 = _tc_layer_mid(a0, a1, xw2, dis_col, b2.reshape(1, _C), W3)
    a0, a1 = segsum(ys3)
    out3, xw4, ys4 = _tc_layer_mid(a0, a1, xw3, dis_col, b3.reshape(1, _C), W3)
    a0, a1 = segsum(ys4)
    out4, xw5, ys5 = _tc_layer_mid(a0, a1, xw4, dis_col, b3.reshape(1, _C), W3)
    a0, a1 = segsum(ys5)
    out5 = _tc_layer_last(a0, a1, xw5, dis_col, b3.reshape(1, _C))

    l1s = jnp.stack([
        lin1W[:, 0 * _C:1 * _C].T, lin1W[:, 1 * _C:2 * _C].T,
        lin1W[:, 2 * _C:3 * _C].T, lin1W[:, 3 * _C:4 * _C].T,
        lin1W[:, 4 * _C:5 * _C].T, lin1W[:, 5 * _C:6 * _C].T,
    ])
    la_row = lin1W[:, 6 * _C].reshape(1, 32)
    act_col = action.reshape(_N, 1)
    y = _tc_readout(out1, out2, out3, out4, out5, state, act_col, l1s,
                    la_row, lin1b.reshape(1, 32), lin2W.T,
                    lin2b.reshape(1, 32), lin3W.T, lin3b.reshape(1, 1))
    return y.reshape(_N // 8)


# speedup vs baseline: 22.4058x; 1.0152x over previous
import functools

import jax
import jax.numpy as jnp
from jax import lax
from jax.experimental import pallas as pl
from jax.experimental.pallas import tpu as pltpu
from jax.experimental.pallas import tpu_sc as plsc

_N = 10000
_C = 128
_E = 320000
_NW = 32
_PER_W = _E // _NW
_CH = 128
_NCH = _PER_W // _CH
_TAIL = _PER_W - _NCH * _CH
_RPT = 624

_mesh = plsc.VectorSubcoreMesh(core_axis_name="c", subcore_axis_name="s")


_RING = 13


@functools.partial(
    pl.kernel,
    out_type=jax.ShapeDtypeStruct((2, _N), jnp.float32),
    scratch_types=[
        pltpu.VMEM_SHARED((_N,), jnp.float32),
        pltpu.VMEM((_CH,), jnp.float32),
        pltpu.VMEM((_NCH, 1, _CH), jnp.int32),
        pltpu.VMEM((1, _TAIL), jnp.int32),
        pltpu.SemaphoreType.DMA,
    ],
    mesh=_mesh,
)
def _sc_degree(dstf_hbm, dstt_hbm, zeros_hbm, ones_hbm, out_hbm, shared_deg,
               ones_v, didx3d, didx_t, sem_sc):
    cid = lax.axis_index("c")
    sid = lax.axis_index("s")
    w = cid * 16 + sid

    @pl.when(sid == 0)
    def _():
        pltpu.sync_copy(zeros_hbm, shared_deg)

    pltpu.sync_copy(ones_hbm, ones_v)
    pltpu.sync_copy(dstf_hbm.at[w], didx3d)
    pltpu.sync_copy(dstt_hbm.at[w], didx_t)
    plsc.subcore_barrier()

    def _wait_one():
        pltpu.make_async_copy(ones_v, shared_deg.at[didx3d.at[0, 0]],
                              sem_sc).wait()

    def ring(g, _):
        for k in range(_RING):
            pltpu.async_copy(ones_v, shared_deg.at[didx3d.at[g * _RING + k, 0]],
                             sem_sc, add=True)

        @pl.when(g > 0)
        def _():
            for k in range(_RING):
                _wait_one()

        return 0

    lax.fori_loop(0, _NCH // _RING, ring, 0)
    for k in range(_RING):
        _wait_one()
    pltpu.sync_copy(ones_v.at[pl.ds(0, _TAIL)], shared_deg.at[didx_t.at[0]],
                    add=True)
    plsc.subcore_barrier()

    @pl.when(sid == 0)
    def _():
        pltpu.sync_copy(shared_deg, out_hbm.at[cid])


@functools.partial(
    pl.kernel,
    out_type=jax.ShapeDtypeStruct((2, _N, _C), jnp.float32),
    scratch_types=[
        pltpu.VMEM_SHARED((_N, _C), jnp.float32),
        pltpu.VMEM((_CH,), jnp.int32),
        pltpu.VMEM((_CH,), jnp.int32),
        pltpu.VMEM((_CH,), jnp.int32),
        pltpu.VMEM((_CH,), jnp.int32),
        pltpu.VMEM((_TAIL,), jnp.int32),
        pltpu.VMEM((_TAIL,), jnp.int32),
        pltpu.VMEM((_CH, _C), jnp.float32),
        pltpu.VMEM((_CH, _C), jnp.float32),
        pltpu.SemaphoreType.DMA,
        pltpu.SemaphoreType.DMA,
        pltpu.SemaphoreType.DMA,
        pltpu.SemaphoreType.DMA,
        pltpu.SemaphoreType.DMA,
        pltpu.SemaphoreType.DMA,
    ],
    mesh=_mesh,
)
def _sc_segsum(ys_hbm, src_hbm, dst_hbm, zeros_hbm, out_hbm, shared_acc,
               sidx_a, sidx_b, didx_a, didx_b, sidx_t, didx_t, rows_a,
               rows_b, sem_sa, sem_sb, sem_da, sem_db, sem_a, sem_b):
    cid = lax.axis_index("c")
    sid = lax.axis_index("s")
    w = cid * 16 + sid

    @pl.when(sid < 15)
    def _():
        pltpu.sync_copy(zeros_hbm.at[pl.ds(sid * _RPT, _RPT)],
                        shared_acc.at[pl.ds(sid * _RPT, _RPT)])

    @pl.when(sid == 15)
    def _():
        pltpu.sync_copy(zeros_hbm.at[pl.ds(15 * _RPT, _N - 15 * _RPT)],
                        shared_acc.at[pl.ds(15 * _RPT, _N - 15 * _RPT)])

    base = (cid * 16 + sid) * _PER_W
    plsc.subcore_barrier()

    def _issue_idx(i, sidx, sem_s, didx, sem_d):
        off = base + i * _CH
        pltpu.async_copy(src_hbm.at[pl.ds(off, _CH)], sidx, sem_s)
        pltpu.async_copy(dst_hbm.at[pl.ds(off, _CH)], didx, sem_d)

    def _wait(src_like, dst, sem):
        pltpu.make_async_copy(src_like, dst, sem).wait()

    _issue_idx(0, sidx_a, sem_sa, didx_a, sem_da)
    _issue_idx(1, sidx_b, sem_sb, didx_b, sem_db)
    _wait(src_hbm.at[pl.ds(0, _CH)], sidx_a, sem_sa)
    pltpu.async_copy(ys_hbm.at[sidx_a], rows_a, sem_a)
    _wait(src_hbm.at[pl.ds(0, _CH)], sidx_b, sem_sb)
    pltpu.async_copy(ys_hbm.at[sidx_b], rows_b, sem_b)

    def _step(i, sidx, sem_s, didx, sem_d, rows, sem_g):
        _wait(ys_hbm.at[pl.ds(0, _CH)], rows, sem_g)
        off2 = base + (i + 2) * _CH
        pltpu.async_copy(src_hbm.at[pl.ds(off2, _CH)], sidx, sem_s)
        _wait(src_hbm.at[pl.ds(0, _CH)], didx, sem_d)
        pltpu.sync_copy(rows, shared_acc.at[didx], add=True)
        pltpu.async_copy(dst_hbm.at[pl.ds(off2, _CH)], didx, sem_d)
        _wait(src_hbm.at[pl.ds(0, _CH)], sidx, sem_s)
        pltpu.async_copy(ys_hbm.at[sidx], rows, sem_g)

    def body(i, _):
        @pl.when(i % 2 == 0)
        def _():
            _step(i, sidx_a, sem_sa, didx_a, sem_da, rows_a, sem_a)

        @pl.when(i % 2 == 1)
        def _():
            _step(i, sidx_b, sem_sb, didx_b, sem_db, rows_b, sem_b)

        return 0

    lax.fori_loop(0, _NCH - 2, body, 0)

    def _drain(didx, sem_d, rows, sem_g):
        _wait(ys_hbm.at[pl.ds(0, _CH)], rows, sem_g)
        _wait(src_hbm.at[pl.ds(0, _CH)], didx, sem_d)
        pltpu.sync_copy(rows, shared_acc.at[didx], add=True)

    _drain(didx_a, sem_da, rows_a, sem_a)
    _drain(didx_b, sem_db, rows_b, sem_b)

    o_t = base + _NCH * _CH
    pltpu.async_copy(src_hbm.at[pl.ds(o_t, _TAIL)], sidx_t, sem_sa)
    pltpu.async_copy(dst_hbm.at[pl.ds(o_t, _TAIL)], didx_t, sem_da)
    _wait(src_hbm.at[pl.ds(0, _TAIL)], sidx_t, sem_sa)
    pltpu.async_copy(ys_hbm.at[sidx_t], rows_a.at[pl.ds(0, _TAIL)],
                     sem_a).wait()
    _wait(dst_hbm.at[pl.ds(0, _TAIL)], didx_t, sem_da)
    pltpu.sync_copy(rows_a.at[pl.ds(0, _TAIL)], shared_acc.at[didx_t],
                    add=True)
    plsc.subcore_barrier()

    @pl.when(sid < 15)
    def _():
        pltpu.sync_copy(shared_acc.at[pl.ds(sid * _RPT, _RPT)],
                        out_hbm.at[cid, pl.ds(sid * _RPT, _RPT)])

    @pl.when(sid == 15)
    def _():
        pltpu.sync_copy(shared_acc.at[pl.ds(15 * _RPT, _N - 15 * _RPT)],
                        out_hbm.at[cid, pl.ds(15 * _RPT, _N - 15 * _RPT)])


def _dis_body(deg_ref, out_ref):
    d = deg_ref[0:1, :] + deg_ref[1:2, :] + 1.0
    out_ref[...] = lax.rsqrt(d)


def _tc_dis(deg2):
    return pl.pallas_call(
        _dis_body,
        out_shape=jax.ShapeDtypeStruct((1, _N), jnp.float32),
    )(deg2)


_R = 2000


def _matmul_body(h_ref, w_ref, xw_ref):
    xw_ref[...] = jnp.dot(h_ref[...], w_ref[...],
                          preferred_element_type=jnp.float32)


def _tc_matmul(h, w):
    grid = (_N // _R,)
    return pl.pallas_call(
        _matmul_body,
        grid=grid,
        in_specs=[
            pl.BlockSpec((_R, _C), lambda i: (i, 0)),
            pl.BlockSpec((_C, _C), lambda i: (0, 0)),
        ],
        out_specs=pl.BlockSpec((_R, _C), lambda i: (i, 0)),
        out_shape=jax.ShapeDtypeStruct((_N, _C), jnp.float32),
    )(h, w)


def _scale_body(xw_ref, dis_ref, ys_ref):
    ys_ref[...] = xw_ref[...] * dis_ref[...]


def _tc_scale(xw, dis_col):
    grid = (_N // _R,)
    return pl.pallas_call(
        _scale_body,
        grid=grid,
        in_specs=[
            pl.BlockSpec((_R, _C), lambda i: (i, 0)),
            pl.BlockSpec((_R, 1), lambda i: (i, 0)),
        ],
        out_specs=pl.BlockSpec((_R, _C), lambda i: (i, 0)),
        out_shape=jax.ShapeDtypeStruct((_N, _C), jnp.float32),
    )(xw, dis_col)


def _layer_mid_body(a0_ref, a1_ref, ys_ref, dis_ref, b_ref, w_ref,
                    out_ref, ysn_ref):
    d = dis_ref[...]
    agg = (a0_ref[...] + a1_ref[...] + ys_ref[...]) * d + b_ref[...]
    out = jnp.maximum(agg, 0.0)
    out_ref[...] = out
    xwn = jnp.dot(out, w_ref[...], preferred_element_type=jnp.float32)
    ysn_ref[...] = xwn * d


def _tc_layer_mid(a0, a1, ys, dis_col, b_row, w_next):
    grid = (_N // _R,)
    return pl.pallas_call(
        _layer_mid_body,
        grid=grid,
        in_specs=[
            pl.BlockSpec((_R, _C), lambda i: (i, 0)),
            pl.BlockSpec((_R, _C), lambda i: (i, 0)),
            pl.BlockSpec((_R, _C), lambda i: (i, 0)),
            pl.BlockSpec((_R, 1), lambda i: (i, 0)),
            pl.BlockSpec((1, _C), lambda i: (0, 0)),
            pl.BlockSpec((_C, _C), lambda i: (0, 0)),
        ],
        out_specs=[
            pl.BlockSpec((_R, _C), lambda i: (i, 0)),
            pl.BlockSpec((_R, _C), lambda i: (i, 0)),
        ],
        out_shape=[
            jax.ShapeDtypeStruct((_N, _C), jnp.float32),
            jax.ShapeDtypeStruct((_N, _C), jnp.float32),
        ],
    )(a0, a1, ys, dis_col, b_row, w_next)


def _layer_last_body(a0_ref, a1_ref, ys_ref, dis_ref, b_ref, out_ref):
    d = dis_ref[...]
    agg = (a0_ref[...] + a1_ref[...] + ys_ref[...]) * d + b_ref[...]
    out_ref[...] = jnp.maximum(agg, 0.0)


def _tc_layer_last(a0, a1, ys, dis_col, b_row):
    grid = (_N // _R,)
    return pl.pallas_call(
        _layer_last_body,
        grid=grid,
        in_specs=[
            pl.BlockSpec((_R, _C), lambda i: (i, 0)),
            pl.BlockSpec((_R, _C), lambda i: (i, 0)),
            pl.BlockSpec((_R, _C), lambda i: (i, 0)),
            pl.BlockSpec((_R, 1), lambda i: (i, 0)),
            pl.BlockSpec((1, _C), lambda i: (0, 0)),
        ],
        out_specs=pl.BlockSpec((_R, _C), lambda i: (i, 0)),
        out_shape=jax.ShapeDtypeStruct((_N, _C), jnp.float32),
    )(a0, a1, ys, dis_col, b_row)


_G = _R // 8


def _readout_body(o1_ref, o2_ref, o3_ref, o4_ref, o5_ref, st_ref, act_ref,
                  l1_ref, la_ref, l1b_ref, l2_ref, l2b_ref, l3_ref, l3b_ref,
                  out_ref):
    f32 = jnp.float32
    x1 = jnp.dot(o1_ref[...], l1_ref[0], preferred_element_type=f32)
    x1 += jnp.dot(o2_ref[...], l1_ref[1], preferred_element_type=f32)
    x1 += jnp.dot(o3_ref[...], l1_ref[2], preferred_element_type=f32)
    x1 += jnp.dot(o4_ref[...], l1_ref[3], preferred_element_type=f32)
    x1 += jnp.dot(o5_ref[...], l1_ref[4], preferred_element_type=f32)
    x1 += jnp.dot(st_ref[...], l1_ref[5], preferred_element_type=f32)
    x1 += act_ref[...] * la_ref[...] + l1b_ref[...]
    x1 = jnp.maximum(x1, 0.0)
    x2 = jnp.maximum(
        jnp.dot(x1, l2_ref[...], preferred_element_type=f32) + l2b_ref[...],
        0.0)
    s = jnp.dot(x2, l3_ref[...], preferred_element_type=f32)
    rows = lax.broadcasted_iota(jnp.int32, (_G, _R), 1)
    grp = lax.broadcasted_iota(jnp.int32, (_G, _R), 0)
    pmat = jnp.where(rows // 8 == grp, 1.0, 0.0).astype(f32)
    res = jnp.dot(pmat, s, preferred_element_type=f32) + l3b_ref[...]
    out_ref[...] = res[None]


def _tc_readout(o1, o2, o3, o4, o5, st, act_col, l1s, la_row, l1b_row, l2t,
                l2b_row, l3col, l3b11):
    grid = (_N // _R,)
    rc = pl.BlockSpec((_R, _C), lambda i: (i, 0))
    return pl.pallas_call(
        _readout_body,
        grid=grid,
        in_specs=[
            rc, rc, rc, rc, rc, rc,
            pl.BlockSpec((_R, 1), lambda i: (i, 0)),
            pl.BlockSpec((6, _C, 32), lambda i: (0, 0, 0)),
            pl.BlockSpec((1, 32), lambda i: (0, 0)),
            pl.BlockSpec((1, 32), lambda i: (0, 0)),
            pl.BlockSpec((32, 32), lambda i: (0, 0)),
            pl.BlockSpec((1, 32), lambda i: (0, 0)),
            pl.BlockSpec((32, 1), lambda i: (0, 0)),
            pl.BlockSpec((1, 1), lambda i: (0, 0)),
        ],
        out_specs=pl.BlockSpec((1, _G, 1), lambda i: (i, 0, 0)),
        out_shape=jax.ShapeDtypeStruct((_N // _R, _G, 1), jnp.float32),
    )(o1, o2, o3, o4, o5, st, act_col, l1s, la_row, l1b_row, l2t, l2b_row,
      l3col, l3b11)


def kernel(state, edge_index, action, W1, b1, W2, b2, W3, b3, lin1W, lin1b,
           lin2W, lin2b, lin3W, lin3b):
    src = edge_index[0]
    dst = edge_index[1]
    dstw = dst.reshape(_NW, _PER_W)
    dstf = dstw[:, :_NCH * _CH].reshape(_NW, _NCH, 1, _CH)
    dstt = dstw[:, _NCH * _CH:].reshape(_NW, 1, _TAIL)
    zeros_n = jnp.zeros((_N,), jnp.float32)
    zeros_nc = jnp.zeros((_N, _C), jnp.float32)
    ones_ch = jnp.ones((_CH,), jnp.float32)

    xw1 = _tc_matmul(state, W1)
    deg2 = _sc_degree(dstf, dstt, zeros_n, ones_ch)
    dis_col = _tc_dis(deg2).reshape(_N, 1)

    def segsum(ys):
        acc = _sc_segsum(ys, src, dst, zeros_nc)
        return acc[0], acc[1]

    ys1 = _tc_scale(xw1, dis_col)
    a0, a1 = segsum(ys1)
    out1, ys2 = _tc_layer_mid(a0, a1, ys1, dis_col, b1.reshape(1, _C), W2)
    a0, a1 = segsum(ys2)
    out2, ys3 = _tc_layer_mid(a0, a1, ys2, dis_col, b2.reshape(1, _C), W3)
    a0, a1 = segsum(ys3)
    out3, ys4 = _tc_layer_mid(a0, a1, ys3, dis_col, b3.reshape(1, _C), W3)
    a0, a1 = segsum(ys4)
    out4, ys5 = _tc_layer_mid(a0, a1, ys4, dis_col, b3.reshape(1, _C), W3)
    a0, a1 = segsum(ys5)
    out5 = _tc_layer_last(a0, a1, ys5, dis_col, b3.reshape(1, _C))

    l1s = jnp.stack([
        lin1W[:, 0 * _C:1 * _C].T, lin1W[:, 1 * _C:2 * _C].T,
        lin1W[:, 2 * _C:3 * _C].T, lin1W[:, 3 * _C:4 * _C].T,
        lin1W[:, 4 * _C:5 * _C].T, lin1W[:, 5 * _C:6 * _C].T,
    ])
    la_row = lin1W[:, 6 * _C].reshape(1, 32)
    act_col = action.reshape(_N, 1)
    y = _tc_readout(out1, out2, out3, out4, out5, state, act_col, l1s,
                    la_row, lin1b.reshape(1, 32), lin2W.T,
                    lin2b.reshape(1, 32), lin3W.T, lin3b.reshape(1, 1))
    return y.reshape(_N // 8)


# speedup vs baseline: 22.5306x; 1.0056x over previous
import functools

import jax
import jax.numpy as jnp
from jax import lax
from jax.experimental import pallas as pl
from jax.experimental.pallas import tpu as pltpu
from jax.experimental.pallas import tpu_sc as plsc

_N = 10000
_C = 128
_E = 320000
_NW = 32
_PER_W = _E // _NW
_CH = 128
_NCH = _PER_W // _CH
_TAIL = _PER_W - _NCH * _CH
_RPT = 624

_mesh = plsc.VectorSubcoreMesh(core_axis_name="c", subcore_axis_name="s")


_RING = 13


@functools.partial(
    pl.kernel,
    out_type=jax.ShapeDtypeStruct((2, _N), jnp.float32),
    scratch_types=[
        pltpu.VMEM_SHARED((_N,), jnp.float32),
        pltpu.VMEM((_CH,), jnp.float32),
        pltpu.VMEM((_NCH, 1, _CH), jnp.int32),
        pltpu.VMEM((1, _TAIL), jnp.int32),
        pltpu.SemaphoreType.DMA,
    ],
    mesh=_mesh,
)
def _sc_degree(dstf_hbm, dstt_hbm, zeros_hbm, ones_hbm, out_hbm, shared_deg,
               ones_v, didx3d, didx_t, sem_sc):
    cid = lax.axis_index("c")
    sid = lax.axis_index("s")
    w = cid * 16 + sid

    @pl.when(sid == 0)
    def _():
        pltpu.sync_copy(zeros_hbm, shared_deg)

    pltpu.sync_copy(ones_hbm, ones_v)
    pltpu.sync_copy(dstf_hbm.at[w], didx3d)
    pltpu.sync_copy(dstt_hbm.at[w], didx_t)
    plsc.subcore_barrier()

    def _wait_one():
        pltpu.make_async_copy(ones_v, shared_deg.at[didx3d.at[0, 0]],
                              sem_sc).wait()

    def ring(g, _):
        for k in range(_RING):
            pltpu.async_copy(ones_v, shared_deg.at[didx3d.at[g * _RING + k, 0]],
                             sem_sc, add=True)

        @pl.when(g > 0)
        def _():
            for k in range(_RING):
                _wait_one()

        return 0

    lax.fori_loop(0, _NCH // _RING, ring, 0)
    for k in range(_RING):
        _wait_one()
    pltpu.sync_copy(ones_v.at[pl.ds(0, _TAIL)], shared_deg.at[didx_t.at[0]],
                    add=True)
    plsc.subcore_barrier()

    @pl.when(sid == 0)
    def _():
        pltpu.sync_copy(shared_deg, out_hbm.at[cid])


@functools.partial(
    pl.kernel,
    out_type=jax.ShapeDtypeStruct((2, _N, _C), jnp.float32),
    scratch_types=[
        pltpu.VMEM_SHARED((_N, _C), jnp.float32),
        pltpu.VMEM((_CH,), jnp.int32),
        pltpu.VMEM((_CH,), jnp.int32),
        pltpu.VMEM((_CH,), jnp.int32),
        pltpu.VMEM((_CH,), jnp.int32),
        pltpu.VMEM((_TAIL,), jnp.int32),
        pltpu.VMEM((_TAIL,), jnp.int32),
        pltpu.VMEM((_CH, _C), jnp.float32),
        pltpu.VMEM((_CH, _C), jnp.float32),
        pltpu.SemaphoreType.DMA,
        pltpu.SemaphoreType.DMA,
        pltpu.SemaphoreType.DMA,
        pltpu.SemaphoreType.DMA,
        pltpu.SemaphoreType.DMA,
        pltpu.SemaphoreType.DMA,
    ],
    mesh=_mesh,
)
def _sc_segsum(ys_hbm, src_hbm, dst_hbm, zeros_hbm, out_hbm, shared_acc,
               sidx_a, sidx_b, didx_a, didx_b, sidx_t, didx_t, rows_a,
               rows_b, sem_sa, sem_sb, sem_da, sem_db, sem_a, sem_b):
    cid = lax.axis_index("c")
    sid = lax.axis_index("s")
    w = cid * 16 + sid

    @pl.when(sid < 15)
    def _():
        pltpu.sync_copy(zeros_hbm.at[pl.ds(sid * _RPT, _RPT)],
                        shared_acc.at[pl.ds(sid * _RPT, _RPT)])

    @pl.when(sid == 15)
    def _():
        pltpu.sync_copy(zeros_hbm.at[pl.ds(15 * _RPT, _N - 15 * _RPT)],
                        shared_acc.at[pl.ds(15 * _RPT, _N - 15 * _RPT)])

    base = (cid * 16 + sid) * _PER_W

    def _issue_idx(i, sidx, sem_s, didx, sem_d):
        off = base + i * _CH
        pltpu.async_copy(src_hbm.at[pl.ds(off, _CH)], sidx, sem_s)
        pltpu.async_copy(dst_hbm.at[pl.ds(off, _CH)], didx, sem_d)

    def _wait(src_like, dst, sem):
        pltpu.make_async_copy(src_like, dst, sem).wait()

    _issue_idx(0, sidx_a, sem_sa, didx_a, sem_da)
    _issue_idx(1, sidx_b, sem_sb, didx_b, sem_db)
    _wait(src_hbm.at[pl.ds(0, _CH)], sidx_a, sem_sa)
    pltpu.async_copy(ys_hbm.at[sidx_a], rows_a, sem_a)
    _wait(src_hbm.at[pl.ds(0, _CH)], sidx_b, sem_sb)
    pltpu.async_copy(ys_hbm.at[sidx_b], rows_b, sem_b)
    plsc.subcore_barrier()

    def _step(i, sidx, sem_s, didx, sem_d, rows, sem_g):
        _wait(ys_hbm.at[pl.ds(0, _CH)], rows, sem_g)
        off2 = base + (i + 2) * _CH
        pltpu.async_copy(src_hbm.at[pl.ds(off2, _CH)], sidx, sem_s)
        _wait(src_hbm.at[pl.ds(0, _CH)], didx, sem_d)
        pltpu.sync_copy(rows, shared_acc.at[didx], add=True)
        pltpu.async_copy(dst_hbm.at[pl.ds(off2, _CH)], didx, sem_d)
        _wait(src_hbm.at[pl.ds(0, _CH)], sidx, sem_s)
        pltpu.async_copy(ys_hbm.at[sidx], rows, sem_g)

    def body(i, _):
        @pl.when(i % 2 == 0)
        def _():
            _step(i, sidx_a, sem_sa, didx_a, sem_da, rows_a, sem_a)

        @pl.when(i % 2 == 1)
        def _():
            _step(i, sidx_b, sem_sb, didx_b, sem_db, rows_b, sem_b)

        return 0

    lax.fori_loop(0, _NCH - 2, body, 0)

    def _drain(didx, sem_d, rows, sem_g):
        _wait(ys_hbm.at[pl.ds(0, _CH)], rows, sem_g)
        _wait(src_hbm.at[pl.ds(0, _CH)], didx, sem_d)
        pltpu.sync_copy(rows, shared_acc.at[didx], add=True)

    _drain(didx_a, sem_da, rows_a, sem_a)
    _drain(didx_b, sem_db, rows_b, sem_b)

    o_t = base + _NCH * _CH
    pltpu.async_copy(src_hbm.at[pl.ds(o_t, _TAIL)], sidx_t, sem_sa)
    pltpu.async_copy(dst_hbm.at[pl.ds(o_t, _TAIL)], didx_t, sem_da)
    _wait(src_hbm.at[pl.ds(0, _TAIL)], sidx_t, sem_sa)
    pltpu.async_copy(ys_hbm.at[sidx_t], rows_a.at[pl.ds(0, _TAIL)],
                     sem_a).wait()
    _wait(dst_hbm.at[pl.ds(0, _TAIL)], didx_t, sem_da)
    pltpu.sync_copy(rows_a.at[pl.ds(0, _TAIL)], shared_acc.at[didx_t],
                    add=True)
    plsc.subcore_barrier()

    @pl.when(sid < 15)
    def _():
        pltpu.sync_copy(shared_acc.at[pl.ds(sid * _RPT, _RPT)],
                        out_hbm.at[cid, pl.ds(sid * _RPT, _RPT)])

    @pl.when(sid == 15)
    def _():
        pltpu.sync_copy(shared_acc.at[pl.ds(15 * _RPT, _N - 15 * _RPT)],
                        out_hbm.at[cid, pl.ds(15 * _RPT, _N - 15 * _RPT)])


def _dis_body(deg_ref, out_ref):
    d = deg_ref[0:1, :] + deg_ref[1:2, :] + 1.0
    out_ref[...] = lax.rsqrt(d)


def _tc_dis(deg2):
    return pl.pallas_call(
        _dis_body,
        out_shape=jax.ShapeDtypeStruct((1, _N), jnp.float32),
    )(deg2)


_R = 2000


def _matmul_body(h_ref, w_ref, xw_ref):
    xw_ref[...] = jnp.dot(h_ref[...], w_ref[...],
                          preferred_element_type=jnp.float32)


def _tc_matmul(h, w):
    grid = (_N // _R,)
    return pl.pallas_call(
        _matmul_body,
        grid=grid,
        in_specs=[
            pl.BlockSpec((_R, _C), lambda i: (i, 0)),
            pl.BlockSpec((_C, _C), lambda i: (0, 0)),
        ],
        out_specs=pl.BlockSpec((_R, _C), lambda i: (i, 0)),
        out_shape=jax.ShapeDtypeStruct((_N, _C), jnp.float32),
    )(h, w)


def _scale_body(xw_ref, dis_ref, ys_ref):
    ys_ref[...] = xw_ref[...] * dis_ref[...]


def _tc_scale(xw, dis_col):
    grid = (_N // _R,)
    return pl.pallas_call(
        _scale_body,
        grid=grid,
        in_specs=[
            pl.BlockSpec((_R, _C), lambda i: (i, 0)),
            pl.BlockSpec((_R, 1), lambda i: (i, 0)),
        ],
        out_specs=pl.BlockSpec((_R, _C), lambda i: (i, 0)),
        out_shape=jax.ShapeDtypeStruct((_N, _C), jnp.float32),
    )(xw, dis_col)


def _layer_mid_body(a0_ref, a1_ref, ys_ref, dis_ref, b_ref, w_ref,
                    out_ref, ysn_ref):
    d = dis_ref[...]
    agg = (a0_ref[...] + a1_ref[...] + ys_ref[...]) * d + b_ref[...]
    out = jnp.maximum(agg, 0.0)
    out_ref[...] = out
    xwn = jnp.dot(out, w_ref[...], preferred_element_type=jnp.float32)
    ysn_ref[...] = xwn * d


def _tc_layer_mid(a0, a1, ys, dis_col, b_row, w_next):
    grid = (_N // _R,)
    return pl.pallas_call(
        _layer_mid_body,
        grid=grid,
        in_specs=[
            pl.BlockSpec((_R, _C), lambda i: (i, 0)),
            pl.BlockSpec((_R, _C), lambda i: (i, 0)),
            pl.BlockSpec((_R, _C), lambda i: (i, 0)),
            pl.BlockSpec((_R, 1), lambda i: (i, 0)),
            pl.BlockSpec((1, _C), lambda i: (0, 0)),
            pl.BlockSpec((_C, _C), lambda i: (0, 0)),
        ],
        out_specs=[
            pl.BlockSpec((_R, _C), lambda i: (i, 0)),
            pl.BlockSpec((_R, _C), lambda i: (i, 0)),
        ],
        out_shape=[
            jax.ShapeDtypeStruct((_N, _C), jnp.float32),
            jax.ShapeDtypeStruct((_N, _C), jnp.float32),
        ],
    )(a0, a1, ys, dis_col, b_row, w_next)


def _layer_last_body(a0_ref, a1_ref, ys_ref, dis_ref, b_ref, out_ref):
    d = dis_ref[...]
    agg = (a0_ref[...] + a1_ref[...] + ys_ref[...]) * d + b_ref[...]
    out_ref[...] = jnp.maximum(agg, 0.0)


def _tc_layer_last(a0, a1, ys, dis_col, b_row):
    grid = (_N // _R,)
    return pl.pallas_call(
        _layer_last_body,
        grid=grid,
        in_specs=[
            pl.BlockSpec((_R, _C), lambda i: (i, 0)),
            pl.BlockSpec((_R, _C), lambda i: (i, 0)),
            pl.BlockSpec((_R, _C), lambda i: (i, 0)),
            pl.BlockSpec((_R, 1), lambda i: (i, 0)),
            pl.BlockSpec((1, _C), lambda i: (0, 0)),
        ],
        out_specs=pl.BlockSpec((_R, _C), lambda i: (i, 0)),
        out_shape=jax.ShapeDtypeStruct((_N, _C), jnp.float32),
    )(a0, a1, ys, dis_col, b_row)


_G = _R // 8


def _readout_body(o1_ref, o2_ref, o3_ref, o4_ref, o5_ref, st_ref, act_ref,
                  l1_ref, la_ref, l1b_ref, l2_ref, l2b_ref, l3_ref, l3b_ref,
                  out_ref):
    f32 = jnp.float32
    x1 = jnp.dot(o1_ref[...], l1_ref[0], preferred_element_type=f32)
    x1 += jnp.dot(o2_ref[...], l1_ref[1], preferred_element_type=f32)
    x1 += jnp.dot(o3_ref[...], l1_ref[2], preferred_element_type=f32)
    x1 += jnp.dot(o4_ref[...], l1_ref[3], preferred_element_type=f32)
    x1 += jnp.dot(o5_ref[...], l1_ref[4], preferred_element_type=f32)
    x1 += jnp.dot(st_ref[...], l1_ref[5], preferred_element_type=f32)
    x1 += act_ref[...] * la_ref[...] + l1b_ref[...]
    x1 = jnp.maximum(x1, 0.0)
    x2 = jnp.maximum(
        jnp.dot(x1, l2_ref[...], preferred_element_type=f32) + l2b_ref[...],
        0.0)
    s = jnp.dot(x2, l3_ref[...], preferred_element_type=f32)
    rows = lax.broadcasted_iota(jnp.int32, (_G, _R), 1)
    grp = lax.broadcasted_iota(jnp.int32, (_G, _R), 0)
    pmat = jnp.where(rows // 8 == grp, 1.0, 0.0).astype(f32)
    res = jnp.dot(pmat, s, preferred_element_type=f32) + l3b_ref[...]
    out_ref[...] = res[None]


def _tc_readout(o1, o2, o3, o4, o5, st, act_col, l1s, la_row, l1b_row, l2t,
                l2b_row, l3col, l3b11):
    grid = (_N // _R,)
    rc = pl.BlockSpec((_R, _C), lambda i: (i, 0))
    return pl.pallas_call(
        _readout_body,
        grid=grid,
        in_specs=[
            rc, rc, rc, rc, rc, rc,
            pl.BlockSpec((_R, 1), lambda i: (i, 0)),
            pl.BlockSpec((6, _C, 32), lambda i: (0, 0, 0)),
            pl.BlockSpec((1, 32), lambda i: (0, 0)),
            pl.BlockSpec((1, 32), lambda i: (0, 0)),
            pl.BlockSpec((32, 32), lambda i: (0, 0)),
            pl.BlockSpec((1, 32), lambda i: (0, 0)),
            pl.BlockSpec((32, 1), lambda i: (0, 0)),
            pl.BlockSpec((1, 1), lambda i: (0, 0)),
        ],
        out_specs=pl.BlockSpec((1, _G, 1), lambda i: (i, 0, 0)),
        out_shape=jax.ShapeDtypeStruct((_N // _R, _G, 1), jnp.float32),
    )(o1, o2, o3, o4, o5, st, act_col, l1s, la_row, l1b_row, l2t, l2b_row,
      l3col, l3b11)


def kernel(state, edge_index, action, W1, b1, W2, b2, W3, b3, lin1W, lin1b,
           lin2W, lin2b, lin3W, lin3b):
    src = edge_index[0]
    dst = edge_index[1]
    dstw = dst.reshape(_NW, _PER_W)
    dstf = dstw[:, :_NCH * _CH].reshape(_NW, _NCH, 1, _CH)
    dstt = dstw[:, _NCH * _CH:].reshape(_NW, 1, _TAIL)
    zeros_n = jnp.zeros((_N,), jnp.float32)
    zeros_nc = jnp.zeros((_N, _C), jnp.float32)
    ones_ch = jnp.ones((_CH,), jnp.float32)

    xw1 = _tc_matmul(state, W1)
    deg2 = _sc_degree(dstf, dstt, zeros_n, ones_ch)
    dis_col = _tc_dis(deg2).reshape(_N, 1)

    def segsum(ys):
        acc = _sc_segsum(ys, src, dst, zeros_nc)
        return acc[0], acc[1]

    ys1 = _tc_scale(xw1, dis_col)
    a0, a1 = segsum(ys1)
    out1, ys2 = _tc_layer_mid(a0, a1, ys1, dis_col, b1.reshape(1, _C), W2)
    a0, a1 = segsum(ys2)
    out2, ys3 = _tc_layer_mid(a0, a1, ys2, dis_col, b2.reshape(1, _C), W3)
    a0, a1 = segsum(ys3)
    out3, ys4 = _tc_layer_mid(a0, a1, ys3, dis_col, b3.reshape(1, _C), W3)
    a0, a1 = segsum(ys4)
    out4, ys5 = _tc_layer_mid(a0, a1, ys4, dis_col, b3.reshape(1, _C), W3)
    a0, a1 = segsum(ys5)
    out5 = _tc_layer_last(a0, a1, ys5, dis_col, b3.reshape(1, _C))

    l1s = jnp.stack([
        lin1W[:, 0 * _C:1 * _C].T, lin1W[:, 1 * _C:2 * _C].T,
        lin1W[:, 2 * _C:3 * _C].T, lin1W[:, 3 * _C:4 * _C].T,
        lin1W[:, 4 * _C:5 * _C].T, lin1W[:, 5 * _C:6 * _C].T,
    ])
    la_row = lin1W[:, 6 * _C].reshape(1, 32)
    act_col = action.reshape(_N, 1)
    y = _tc_readout(out1, out2, out3, out4, out5, state, act_col, l1s,
                    la_row, lin1b.reshape(1, 32), lin2W.T,
                    lin2b.reshape(1, 32), lin3W.T, lin3b.reshape(1, 1))
    return y.reshape(_N // 8)
